# Initial kernel scaffold; baseline (speedup 1.0000x reference)
import jax
import jax.numpy as jnp
from jax.experimental import pallas as pl


def kernel(location, cls_pred, box_pred, center_pred, image_sizes):
    raise NotImplementedError("write your pallas kernel here")


# speedup vs baseline: 4.4051x; 4.4051x over previous
import functools
import jax
import jax.numpy as jnp
from jax import lax
from jax.experimental import pallas as pl
from jax.experimental.pallas import tpu as pltpu
from jax.experimental.pallas import tpu_sc as plsc

H = 100
W = 200
HW = H * W
C = 80
B = 8
N = HW * C
TOPN = 1000
NTILES = 32
TPB = NTILES // B
RNG = N // TPB
CHUNK = 4000
NCHUNK = RNG // CHUNK
CAP = 1024
PSC = 2000

@functools.lru_cache(maxsize=None)
def _mesh():
    return plsc.VectorSubcoreMesh(core_axis_name="c", subcore_axis_name="s")


def _wid():
    return lax.axis_index("s") * 2 + lax.axis_index("c")


def _score_body(cls_ref, ctr_ref, out_ref):
    cls = jax.nn.sigmoid(cls_ref[0].reshape(C, HW))
    ctr = jax.nn.sigmoid(ctr_ref[0].reshape(1, HW))
    s = jnp.where(cls > 0.05, cls * ctr, 0.0)
    out_ref[0] = s.T


def _scores(cls_pred, center_pred):
    return pl.pallas_call(
        _score_body,
        grid=(B,),
        in_specs=[
            pl.BlockSpec((1, C, H, W), lambda b: (b, 0, 0, 0)),
            pl.BlockSpec((1, 1, H, W), lambda b: (b, 0, 0, 0)),
        ],
        out_specs=pl.BlockSpec((1, HW, C), lambda b: (b, 0, 0)),
        out_shape=jax.ShapeDtypeStruct((B, HW, C), jnp.float32),
    )(cls_pred, center_pred)


def _hist_hi_body(scores_hbm, hist_out, chunk_v, hist_v, sem):
    wid = _wid()
    base = wid * RNG

    def zero(i, _):
        hist_v[pl.ds(i * 16, 16)] = jnp.zeros((16,), jnp.int32)
        return 0
    lax.fori_loop(0, 16384 // 16, zero, 0)

    ones = jnp.ones((16,), jnp.int32)

    def chunk(k, _):
        pltpu.sync_copy(scores_hbm.at[pl.ds(base + k * CHUNK, CHUNK)], chunk_v)

        def inner(i, _):
            v = chunk_v[pl.ds(i * 16, 16)]
            bits = lax.bitcast_convert_type(v, jnp.int32)
            hi = lax.shift_right_logical(bits, 16)
            plsc.addupdate_scatter(hist_v, [hi], ones, mask=bits >= 0)
            return 0
        lax.fori_loop(0, CHUNK // 16, inner, 0)
        return 0
    lax.fori_loop(0, NCHUNK, chunk, 0)
    pltpu.sync_copy(hist_v, hist_out.at[wid])


@functools.lru_cache(maxsize=None)
def _hist_hi():
    return functools.partial(
        pl.kernel, mesh=_mesh(),
        compiler_params=pltpu.CompilerParams(
            needs_layout_passes=False, use_tc_tiling_on_sc=False),
        out_type=jax.ShapeDtypeStruct((NTILES, 16384), jnp.int32),
        scratch_types=[
            pltpu.VMEM((CHUNK,), jnp.float32),
            pltpu.VMEM((16384,), jnp.int32),
            pltpu.SemaphoreType.DMA,
        ],
    )(_hist_hi_body)


def _sel_lane(vec16, lane):
    msk = lax.iota(jnp.int32, 16) == lane
    return jnp.max(jnp.where(msk, vec16, jnp.int32(-2147483648)))


def _hist_lo_body(scores_hbm, bstar_hbm, hist_out, chunk_v, hist_v, bst_v, sem):
    wid = _wid()
    base = wid * RNG
    b = wid // TPB
    pltpu.sync_copy(bstar_hbm, bst_v)
    bstar = _sel_lane(bst_v[...], b)

    def zero(i, _):
        hist_v[pl.ds(i * 16, 16)] = jnp.zeros((16,), jnp.int32)
        return 0
    lax.fori_loop(0, 65536 // 16, zero, 0)

    ones = jnp.ones((16,), jnp.int32)

    def chunk(k, _):
        pltpu.sync_copy(scores_hbm.at[pl.ds(base + k * CHUNK, CHUNK)], chunk_v)

        def inner(i, _):
            v = chunk_v[pl.ds(i * 16, 16)]
            bits = lax.bitcast_convert_type(v, jnp.int32)
            hi = lax.shift_right_logical(bits, 16)
            lo = jnp.bitwise_and(bits, jnp.int32(0xFFFF))
            plsc.addupdate_scatter(hist_v, [lo], ones, mask=hi == bstar)
            return 0
        lax.fori_loop(0, CHUNK // 16, inner, 0)
        return 0
    lax.fori_loop(0, NCHUNK, chunk, 0)
    pltpu.sync_copy(hist_v, hist_out.at[wid])


@functools.lru_cache(maxsize=None)
def _hist_lo():
    return functools.partial(
        pl.kernel, mesh=_mesh(),
        compiler_params=pltpu.CompilerParams(
            needs_layout_passes=False, use_tc_tiling_on_sc=False),
        out_type=jax.ShapeDtypeStruct((NTILES, 65536), jnp.int32),
        scratch_types=[
            pltpu.VMEM((CHUNK,), jnp.float32),
            pltpu.VMEM((65536,), jnp.int32),
            pltpu.VMEM((16,), jnp.int32),
            pltpu.SemaphoreType.DMA,
        ],
    )(_hist_lo_body)


def _collect_body(scores_hbm, vstar_hbm, gt_out, eq_out, cnt_out,
                  chunk_v, gt_v, eq_v, vst_v, cnt_v, sem):
    wid = _wid()
    base = wid * RNG
    b = wid // TPB
    ibase = (wid % TPB) * RNG
    pltpu.sync_copy(vstar_hbm, vst_v)
    vstar = _sel_lane(vst_v[...], b)
    lanes = lax.iota(jnp.int32, 16)

    def zero(i, _):
        gt_v[pl.ds(i * 16, 16)] = jnp.zeros((16,), jnp.int32)
        eq_v[pl.ds(i * 16, 16)] = jnp.zeros((16,), jnp.int32)
        return 0
    lax.fori_loop(0, CAP // 16, zero, 0)

    def chunk(k, carry):
        cgt, ceq = carry
        pltpu.sync_copy(scores_hbm.at[pl.ds(base + k * CHUNK, CHUNK)], chunk_v)

        def inner(i, carry2):
            cgt2, ceq2 = carry2
            v = chunk_v[pl.ds(i * 16, 16)]
            bits = lax.bitcast_convert_type(v, jnp.int32)
            gidx = (ibase + k * CHUNK + i * 16) + lanes
            gm = bits > vstar
            em = bits == vstar
            gpos = cgt2 + plsc.cumsum(gm.astype(jnp.int32)) - 1
            plsc.store_scatter(gt_v, [gpos], gidx, mask=gm & (gpos < CAP))
            epos = ceq2 + plsc.cumsum(em.astype(jnp.int32)) - 1
            plsc.store_scatter(eq_v, [epos], gidx, mask=em & (epos < CAP))
            return (cgt2 + jnp.sum(gm.astype(jnp.int32)),
                    ceq2 + jnp.sum(em.astype(jnp.int32)))
        return lax.fori_loop(0, CHUNK // 16, inner, (cgt, ceq))

    cgt, ceq = lax.fori_loop(0, NCHUNK, chunk, (jnp.int32(0), jnp.int32(0)))
    cnt_v[...] = jnp.where(lanes == 0, cgt, jnp.where(lanes == 1, ceq, 0))
    pltpu.sync_copy(gt_v, gt_out.at[wid])
    pltpu.sync_copy(eq_v, eq_out.at[wid])
    pltpu.sync_copy(cnt_v, cnt_out.at[wid])


@functools.lru_cache(maxsize=None)
def _collect():
    return functools.partial(
        pl.kernel, mesh=_mesh(),
        compiler_params=pltpu.CompilerParams(
            needs_layout_passes=False, use_tc_tiling_on_sc=False),
        out_type=(
            jax.ShapeDtypeStruct((NTILES, CAP), jnp.int32),
            jax.ShapeDtypeStruct((NTILES, CAP), jnp.int32),
            jax.ShapeDtypeStruct((NTILES, 16), jnp.int32),
        ),
        scratch_types=[
            pltpu.VMEM((CHUNK,), jnp.float32),
            pltpu.VMEM((CAP,), jnp.int32),
            pltpu.VMEM((CAP,), jnp.int32),
            pltpu.VMEM((16,), jnp.int32),
            pltpu.VMEM((16,), jnp.int32),
            pltpu.SemaphoreType.DMA,
        ],
    )(_collect_body)


SLOTS = CAP // TPB
NF = 16


def _gather_body(lists_hbm, src_hbm, scores_hbm, loc_hbm, box_hbm, out_hbm,
                 src_v, cand_v, addr_v, fld_v, sem):
    wid = _wid()
    b = wid // TPB
    q = wid % TPB
    pltpu.sync_copy(src_hbm.at[pl.ds(b * CAP + q * SLOTS, SLOTS)], src_v)
    pltpu.async_copy(lists_hbm.at[src_v], cand_v, sem).wait()

    def addrs(i, field, fn):
        def body(j, _):
            cand = cand_v[pl.ds(j * 16, 16)]
            addr_v[pl.ds(j * 16, 16)] = fn(cand)
            return 0
        lax.fori_loop(0, SLOTS // 16, body, 0)

    addrs(0, 0, lambda cand: b * N + cand)
    pltpu.async_copy(scores_hbm.at[addr_v], fld_v.at[0], sem).wait()
    addrs(0, 1, lambda cand: 2 * (cand // C))
    pltpu.async_copy(loc_hbm.at[addr_v], fld_v.at[1], sem).wait()
    addrs(0, 2, lambda cand: 2 * (cand // C) + 1)
    pltpu.async_copy(loc_hbm.at[addr_v], fld_v.at[2], sem).wait()
    for f in range(4):
        addrs(0, 3 + f, lambda cand, f=f: (b * 4 + f) * HW + cand // C)
        pltpu.async_copy(box_hbm.at[addr_v], fld_v.at[3 + f], sem).wait()

    def cls_body(j, _):
        cand = cand_v[pl.ds(j * 16, 16)]
        c = cand - (cand // C) * C
        fld_v[7, pl.ds(j * 16, 16)] = (c + 1).astype(jnp.float32)
        fld_v[8, pl.ds(j * 16, 16)] = cand.astype(jnp.float32)
        return 0
    lax.fori_loop(0, SLOTS // 16, cls_body, 0)

    for f in range(9):
        pltpu.sync_copy(fld_v.at[f], out_hbm.at[b, f, pl.ds(q * SLOTS, SLOTS)])


@functools.lru_cache(maxsize=None)
def _gather():
    return functools.partial(
        pl.kernel, mesh=_mesh(),
        compiler_params=pltpu.CompilerParams(
            needs_layout_passes=False, use_tc_tiling_on_sc=False),
        out_type=jax.ShapeDtypeStruct((B, NF, CAP), jnp.float32),
        scratch_types=[
            pltpu.VMEM((SLOTS,), jnp.int32),
            pltpu.VMEM((SLOTS,), jnp.int32),
            pltpu.VMEM((SLOTS,), jnp.int32),
            pltpu.VMEM((NF, SLOTS), jnp.float32),
            pltpu.SemaphoreType.DMA,
        ],
    )(_gather_body)


RT = 128


def _decode(cand, jbase, nlanes, wimg, himg):
    j_i32 = jbase + lax.broadcasted_iota(jnp.int32, (nlanes,), 0)
    jf = j_i32.astype(jnp.float32)
    padm = j_i32 >= TOPN
    score = jnp.where(padm, -1.0, cand[0])
    ridx = jnp.where(padm, 2.0e6 + jf, cand[8])
    s = jnp.where(padm, 0.0, cand[0])
    x1 = jnp.clip(cand[1] - cand[3], 0.0, wimg - 1.0)
    y1 = jnp.clip(cand[2] - cand[4], 0.0, himg - 1.0)
    x2 = jnp.clip(cand[1] + cand[5], 0.0, wimg - 1.0)
    y2 = jnp.clip(cand[2] + cand[6], 0.0, himg - 1.0)
    det = jnp.sqrt(jnp.maximum(s, 1e-12))
    valid = s > 0.0
    off = cand[7] * (jnp.maximum(wimg, himg) + 1.0)
    area = jnp.maximum(x2 - x1, 0.0) * jnp.maximum(y2 - y1, 0.0)
    return dict(score=score, ridx=ridx, det=det, valid=valid, x1=x1, y1=y1,
                x2=x2, y2=y2, ox1=x1 + off, oy1=y1 + off, ox2=x2 + off,
                oy2=y2 + off, area=area, cls=cand[7])


def _nms_a_body(sz_ref, cand_ref, ctile_ref, out_ref):
    t = pl.program_id(1)
    wimg = sz_ref[0, 1]
    himg = sz_ref[0, 0]
    F = _decode(cand_ref[0], 0, CAP, wimg, himg)
    T = _decode(ctile_ref[0], t * RT, RT, wimg, himg)

    before = (F["score"][None, :] > T["score"][:, None]) | (
        (F["score"][None, :] == T["score"][:, None])
        & (F["ridx"][None, :] < T["ridx"][:, None]))
    rank = jnp.sum(before.astype(jnp.int32), axis=1).astype(jnp.float32)

    ltx = jnp.maximum(T["ox1"][:, None], F["ox1"][None, :])
    lty = jnp.maximum(T["oy1"][:, None], F["oy1"][None, :])
    rbx = jnp.minimum(T["ox2"][:, None], F["ox2"][None, :])
    rby = jnp.minimum(T["oy2"][:, None], F["oy2"][None, :])
    ww = jnp.clip(rbx - ltx, 0.0, None)
    hh = jnp.clip(rby - lty, 0.0, None)
    inter = ww * hh
    union = T["area"][:, None] + F["area"][None, :] - inter
    iou = inter / jnp.maximum(union, 1e-6)
    hot = ((iou > 0.6) & (F["det"][None, :] > T["det"][:, None])
           & F["valid"][None, :])
    sup = jnp.any(hot, axis=1)

    keep = jnp.where(T["valid"] & (~sup), T["det"], 0.0)
    out_ref[0] = jnp.stack(
        [keep, rank, T["x1"], T["y1"], T["x2"], T["y2"],
         T["cls"], keep], axis=0)


def _nms_b_body(mid_ref, out_ref):
    mid = mid_ref[0]
    keep = mid[0]
    rankf = mid[1]
    franks = []
    for t in range(CAP // RT):
        sl = slice(t * RT, (t + 1) * RT)
        b2 = (keep[None, :] > keep[sl][:, None]) | (
            (keep[None, :] == keep[sl][:, None])
            & (rankf[None, :] < rankf[sl][:, None]))
        franks.append(jnp.sum(b2.astype(jnp.int32), axis=1))
    frank = jnp.concatenate(franks)
    iota_r = lax.broadcasted_iota(jnp.int32, (128, CAP), 0)
    Q = (frank[None, :] == iota_r).astype(jnp.float32)
    G = jnp.stack([mid[2], mid[3], mid[4], mid[5], keep, mid[6],
                   keep, keep], axis=1)
    out_ref[0] = jnp.dot(Q, G, precision=lax.Precision.HIGHEST)


def _nms(imgsz_f32, cand):
    mid = pl.pallas_call(
        _nms_a_body,
        grid=(B, CAP // RT),
        in_specs=[
            pl.BlockSpec(memory_space=pltpu.SMEM),
            pl.BlockSpec((1, NF, CAP), lambda b, t: (b, 0, 0)),
            pl.BlockSpec((1, NF, RT), lambda b, t: (b, 0, t)),
        ],
        out_specs=pl.BlockSpec((1, 8, RT), lambda b, t: (b, 0, t)),
        out_shape=jax.ShapeDtypeStruct((B, 8, CAP), jnp.float32),
    )(imgsz_f32, cand, cand)
    return pl.pallas_call(
        _nms_b_body,
        grid=(B,),
        in_specs=[pl.BlockSpec((1, 8, CAP), lambda b: (b, 0, 0))],
        out_specs=pl.BlockSpec((1, 128, 8), lambda b: (b, 0, 0)),
        out_shape=jax.ShapeDtypeStruct((B, 128, 8), jnp.float32),
    )(mid)


def _excl_cumsum(x, axis):
    c = jnp.cumsum(x, axis=axis)
    return c - x


def _pick_threshold(counts, need):
    nb = counts.shape[1]
    cum = jnp.cumsum(counts[:, ::-1], axis=1)[:, ::-1]
    ge = cum >= need[:, None]
    idx = jnp.max(jnp.where(ge, jnp.arange(nb, dtype=jnp.int32)[None, :], -1),
                  axis=1)
    cum_pad = jnp.concatenate([cum, jnp.zeros((B, 1), jnp.int32)], axis=1)
    ngt = jnp.take_along_axis(cum_pad, (idx + 1)[:, None], axis=1)[:, 0]
    return idx, ngt


def _build_src(cnts, m):
    ngt = cnts[:, 0].reshape(B, TPB)
    neq = jnp.minimum(cnts[:, 1].reshape(B, TPB), CAP)
    gt_off = _excl_cumsum(ngt, 1)
    total_gt = jnp.sum(ngt, axis=1)
    eq_take = jnp.clip(m[:, None] - _excl_cumsum(neq, 1), 0, neq)
    eq_off = _excl_cumsum(eq_take, 1)
    j = jnp.arange(CAP, dtype=jnp.int32)[None, :]
    j2 = j - total_gt[:, None]
    bb = jnp.arange(B, dtype=jnp.int32)[:, None]
    src = jnp.zeros((B, CAP), jnp.int32)
    for t in range(TPB):
        go = gt_off[:, t:t + 1]
        in_g = (j >= go) & (j < go + ngt[:, t:t + 1])
        src = jnp.where(in_g, (bb * TPB + t) * CAP + (j - go), src)
        eo = eq_off[:, t:t + 1]
        in_e = (j2 >= eo) & (j2 < eo + eq_take[:, t:t + 1]) & (j < TOPN)
        src = jnp.where(in_e, NTILES * CAP + (bb * TPB + t) * CAP + (j2 - eo),
                        src)
    return src


def kernel(location, cls_pred, box_pred, center_pred, image_sizes):
    scores = _scores(cls_pred, center_pred)
    scores_flat = scores.reshape(B * N)

    need = jnp.full((B,), TOPN, jnp.int32)
    hist1 = _hist_hi()(scores_flat)
    h1 = jnp.sum(hist1.reshape(B, TPB, 16384), axis=1)
    bstar, ngt_hi = _pick_threshold(h1, need)

    bstar16 = jnp.zeros((16,), jnp.int32).at[:B].set(bstar)
    hist2 = _hist_lo()(scores_flat, bstar16)
    h2 = jnp.sum(hist2.reshape(B, TPB, 65536), axis=1)
    vlo, ngt_lo = _pick_threshold(h2, need - ngt_hi)
    vstar = jnp.left_shift(bstar, 16) | vlo
    n_gt = ngt_hi + ngt_lo
    m = need - n_gt

    vstar16 = jnp.zeros((16,), jnp.int32).at[:B].set(vstar)
    gt_idx, eq_idx, cnts = _collect()(scores_flat, vstar16)
    src = _build_src(cnts, m)

    lists_cat = jnp.concatenate([gt_idx.reshape(-1), eq_idx.reshape(-1)])
    cand = _gather()(lists_cat, src.reshape(-1), scores_flat,
                     location.reshape(-1), box_pred.reshape(-1))

    imgsz = image_sizes.astype(jnp.float32).reshape(1, 2)
    raw = _nms(imgsz, cand)
    detections = raw[:, :100, :5]
    labels = raw[:, :100, 5].astype(jnp.int32)
    return detections, labels


# speedup vs baseline: 4.4686x; 1.0144x over previous
import functools
import jax
import jax.numpy as jnp
from jax import lax
from jax.experimental import pallas as pl
from jax.experimental.pallas import tpu as pltpu
from jax.experimental.pallas import tpu_sc as plsc

H = 100
W = 200
HW = H * W
C = 80
B = 8
N = HW * C
TOPN = 1000
NTILES = 32
TPB = NTILES // B
RNG = N // TPB
CHUNK = 4000
NCHUNK = RNG // CHUNK
UNROLL = 10
CAP = 1024
PSC = 2000

@functools.lru_cache(maxsize=None)
def _mesh():
    return plsc.VectorSubcoreMesh(core_axis_name="c", subcore_axis_name="s")


def _wid():
    return lax.axis_index("s") * 2 + lax.axis_index("c")


def _score_body(cls_ref, ctr_ref, out_ref):
    cls = jax.nn.sigmoid(cls_ref[0].reshape(C, HW))
    ctr = jax.nn.sigmoid(ctr_ref[0].reshape(1, HW))
    s = jnp.where(cls > 0.05, cls * ctr, 0.0)
    out_ref[0] = s.T


def _scores(cls_pred, center_pred):
    return pl.pallas_call(
        _score_body,
        grid=(B,),
        in_specs=[
            pl.BlockSpec((1, C, H, W), lambda b: (b, 0, 0, 0)),
            pl.BlockSpec((1, 1, H, W), lambda b: (b, 0, 0, 0)),
        ],
        out_specs=pl.BlockSpec((1, HW, C), lambda b: (b, 0, 0)),
        out_shape=jax.ShapeDtypeStruct((B, HW, C), jnp.float32),
    )(cls_pred, center_pred)


def _hist_hi_body(scores_hbm, hist_out, chunk_v, hist_v, sem):
    wid = _wid()
    base = wid * RNG

    def zero(i, _):
        hist_v[pl.ds(i * 16, 16)] = jnp.zeros((16,), jnp.int32)
        return 0
    lax.fori_loop(0, 16384 // 16, zero, 0)

    ones = jnp.ones((16,), jnp.int32)

    def chunk(k, _):
        pltpu.sync_copy(scores_hbm.at[pl.ds(base + k * CHUNK, CHUNK)], chunk_v)

        def inner(i, _):
            for u in range(UNROLL):
                v = chunk_v[pl.ds((i * UNROLL + u) * 16, 16)]
                bits = lax.bitcast_convert_type(v, jnp.int32)
                hi = lax.shift_right_logical(bits, 16)
                plsc.addupdate_scatter(hist_v, [hi], ones, mask=bits >= 0)
            return 0
        lax.fori_loop(0, CHUNK // 16 // UNROLL, inner, 0)
        return 0
    lax.fori_loop(0, NCHUNK, chunk, 0)
    pltpu.sync_copy(hist_v, hist_out.at[wid])


@functools.lru_cache(maxsize=None)
def _hist_hi():
    return functools.partial(
        pl.kernel, mesh=_mesh(),
        compiler_params=pltpu.CompilerParams(
            needs_layout_passes=False, use_tc_tiling_on_sc=False),
        out_type=jax.ShapeDtypeStruct((NTILES, 16384), jnp.int32),
        scratch_types=[
            pltpu.VMEM((CHUNK,), jnp.float32),
            pltpu.VMEM((16384,), jnp.int32),
            pltpu.SemaphoreType.DMA,
        ],
    )(_hist_hi_body)


def _sel_lane(vec16, lane):
    msk = lax.iota(jnp.int32, 16) == lane
    return jnp.max(jnp.where(msk, vec16, jnp.int32(-2147483648)))


def _hist_lo_body(scores_hbm, bstar_hbm, hist_out, chunk_v, hist_v, bst_v, sem):
    wid = _wid()
    base = wid * RNG
    b = wid // TPB
    pltpu.sync_copy(bstar_hbm, bst_v)
    bstar = _sel_lane(bst_v[...], b)

    def zero(i, _):
        hist_v[pl.ds(i * 16, 16)] = jnp.zeros((16,), jnp.int32)
        return 0
    lax.fori_loop(0, 65536 // 16, zero, 0)

    ones = jnp.ones((16,), jnp.int32)

    def chunk(k, _):
        pltpu.sync_copy(scores_hbm.at[pl.ds(base + k * CHUNK, CHUNK)], chunk_v)

        def inner(i, _):
            for u in range(UNROLL):
                v = chunk_v[pl.ds((i * UNROLL + u) * 16, 16)]
                bits = lax.bitcast_convert_type(v, jnp.int32)
                hi = lax.shift_right_logical(bits, 16)
                lo = jnp.bitwise_and(bits, jnp.int32(0xFFFF))
                plsc.addupdate_scatter(hist_v, [lo], ones, mask=hi == bstar)
            return 0
        lax.fori_loop(0, CHUNK // 16 // UNROLL, inner, 0)
        return 0
    lax.fori_loop(0, NCHUNK, chunk, 0)
    pltpu.sync_copy(hist_v, hist_out.at[wid])


@functools.lru_cache(maxsize=None)
def _hist_lo():
    return functools.partial(
        pl.kernel, mesh=_mesh(),
        compiler_params=pltpu.CompilerParams(
            needs_layout_passes=False, use_tc_tiling_on_sc=False),
        out_type=jax.ShapeDtypeStruct((NTILES, 65536), jnp.int32),
        scratch_types=[
            pltpu.VMEM((CHUNK,), jnp.float32),
            pltpu.VMEM((65536,), jnp.int32),
            pltpu.VMEM((16,), jnp.int32),
            pltpu.SemaphoreType.DMA,
        ],
    )(_hist_lo_body)


def _collect_body(scores_hbm, vstar_hbm, gt_out, eq_out, cnt_out,
                  chunk_v, gt_v, eq_v, vst_v, cnt_v, sem):
    wid = _wid()
    base = wid * RNG
    b = wid // TPB
    ibase = (wid % TPB) * RNG
    pltpu.sync_copy(vstar_hbm, vst_v)
    vstar = _sel_lane(vst_v[...], b)
    lanes = lax.iota(jnp.int32, 16)

    def zero(i, _):
        gt_v[pl.ds(i * 16, 16)] = jnp.zeros((16,), jnp.int32)
        eq_v[pl.ds(i * 16, 16)] = jnp.zeros((16,), jnp.int32)
        return 0
    lax.fori_loop(0, CAP // 16, zero, 0)

    def chunk(k, carry):
        cgt, ceq = carry
        pltpu.sync_copy(scores_hbm.at[pl.ds(base + k * CHUNK, CHUNK)], chunk_v)

        def inner(i, carry2):
            cgt2, ceq2 = carry2
            for u in range(UNROLL):
                v = chunk_v[pl.ds((i * UNROLL + u) * 16, 16)]
                bits = lax.bitcast_convert_type(v, jnp.int32)
                gidx = (ibase + k * CHUNK + (i * UNROLL + u) * 16) + lanes
                gm = bits > vstar
                em = bits == vstar
                gpos = cgt2 + plsc.cumsum(gm.astype(jnp.int32)) - 1
                plsc.store_scatter(gt_v, [gpos], gidx, mask=gm & (gpos < CAP))
                epos = ceq2 + plsc.cumsum(em.astype(jnp.int32)) - 1
                plsc.store_scatter(eq_v, [epos], gidx, mask=em & (epos < CAP))
                cgt2 = cgt2 + jnp.sum(gm.astype(jnp.int32))
                ceq2 = ceq2 + jnp.sum(em.astype(jnp.int32))
            return (cgt2, ceq2)
        return lax.fori_loop(0, CHUNK // 16 // UNROLL, inner, (cgt, ceq))

    cgt, ceq = lax.fori_loop(0, NCHUNK, chunk, (jnp.int32(0), jnp.int32(0)))
    cnt_v[...] = jnp.where(lanes == 0, cgt, jnp.where(lanes == 1, ceq, 0))
    pltpu.sync_copy(gt_v, gt_out.at[wid])
    pltpu.sync_copy(eq_v, eq_out.at[wid])
    pltpu.sync_copy(cnt_v, cnt_out.at[wid])


@functools.lru_cache(maxsize=None)
def _collect():
    return functools.partial(
        pl.kernel, mesh=_mesh(),
        compiler_params=pltpu.CompilerParams(
            needs_layout_passes=False, use_tc_tiling_on_sc=False),
        out_type=(
            jax.ShapeDtypeStruct((NTILES, CAP), jnp.int32),
            jax.ShapeDtypeStruct((NTILES, CAP), jnp.int32),
            jax.ShapeDtypeStruct((NTILES, 16), jnp.int32),
        ),
        scratch_types=[
            pltpu.VMEM((CHUNK,), jnp.float32),
            pltpu.VMEM((CAP,), jnp.int32),
            pltpu.VMEM((CAP,), jnp.int32),
            pltpu.VMEM((16,), jnp.int32),
            pltpu.VMEM((16,), jnp.int32),
            pltpu.SemaphoreType.DMA,
        ],
    )(_collect_body)


SLOTS = CAP // TPB
NF = 16


def _gather_body(lists_hbm, src_hbm, scores_hbm, loc_hbm, box_hbm, out_hbm,
                 src_v, cand_v, addr_v, fld_v, sem):
    wid = _wid()
    b = wid // TPB
    q = wid % TPB
    pltpu.sync_copy(src_hbm.at[pl.ds(b * CAP + q * SLOTS, SLOTS)], src_v)
    pltpu.async_copy(lists_hbm.at[src_v], cand_v, sem).wait()

    def addrs(i, field, fn):
        def body(j, _):
            cand = cand_v[pl.ds(j * 16, 16)]
            addr_v[pl.ds(j * 16, 16)] = fn(cand)
            return 0
        lax.fori_loop(0, SLOTS // 16, body, 0)

    addrs(0, 0, lambda cand: b * N + cand)
    pltpu.async_copy(scores_hbm.at[addr_v], fld_v.at[0], sem).wait()
    addrs(0, 1, lambda cand: 2 * (cand // C))
    pltpu.async_copy(loc_hbm.at[addr_v], fld_v.at[1], sem).wait()
    addrs(0, 2, lambda cand: 2 * (cand // C) + 1)
    pltpu.async_copy(loc_hbm.at[addr_v], fld_v.at[2], sem).wait()
    for f in range(4):
        addrs(0, 3 + f, lambda cand, f=f: (b * 4 + f) * HW + cand // C)
        pltpu.async_copy(box_hbm.at[addr_v], fld_v.at[3 + f], sem).wait()

    def cls_body(j, _):
        cand = cand_v[pl.ds(j * 16, 16)]
        c = cand - (cand // C) * C
        fld_v[7, pl.ds(j * 16, 16)] = (c + 1).astype(jnp.float32)
        fld_v[8, pl.ds(j * 16, 16)] = cand.astype(jnp.float32)
        return 0
    lax.fori_loop(0, SLOTS // 16, cls_body, 0)

    for f in range(9):
        pltpu.sync_copy(fld_v.at[f], out_hbm.at[b, f, pl.ds(q * SLOTS, SLOTS)])


@functools.lru_cache(maxsize=None)
def _gather():
    return functools.partial(
        pl.kernel, mesh=_mesh(),
        compiler_params=pltpu.CompilerParams(
            needs_layout_passes=False, use_tc_tiling_on_sc=False),
        out_type=jax.ShapeDtypeStruct((B, NF, CAP), jnp.float32),
        scratch_types=[
            pltpu.VMEM((SLOTS,), jnp.int32),
            pltpu.VMEM((SLOTS,), jnp.int32),
            pltpu.VMEM((SLOTS,), jnp.int32),
            pltpu.VMEM((NF, SLOTS), jnp.float32),
            pltpu.SemaphoreType.DMA,
        ],
    )(_gather_body)


RT = 128


def _decode(cand, jbase, nlanes, wimg, himg):
    j_i32 = jbase + lax.broadcasted_iota(jnp.int32, (nlanes,), 0)
    jf = j_i32.astype(jnp.float32)
    padm = j_i32 >= TOPN
    score = jnp.where(padm, -1.0, cand[0])
    ridx = jnp.where(padm, 2.0e6 + jf, cand[8])
    s = jnp.where(padm, 0.0, cand[0])
    x1 = jnp.clip(cand[1] - cand[3], 0.0, wimg - 1.0)
    y1 = jnp.clip(cand[2] - cand[4], 0.0, himg - 1.0)
    x2 = jnp.clip(cand[1] + cand[5], 0.0, wimg - 1.0)
    y2 = jnp.clip(cand[2] + cand[6], 0.0, himg - 1.0)
    det = jnp.sqrt(jnp.maximum(s, 1e-12))
    valid = s > 0.0
    off = cand[7] * (jnp.maximum(wimg, himg) + 1.0)
    area = jnp.maximum(x2 - x1, 0.0) * jnp.maximum(y2 - y1, 0.0)
    return dict(score=score, ridx=ridx, det=det, valid=valid, x1=x1, y1=y1,
                x2=x2, y2=y2, ox1=x1 + off, oy1=y1 + off, ox2=x2 + off,
                oy2=y2 + off, area=area, cls=cand[7])


def _nms_a_body(sz_ref, cand_ref, ctile_ref, out_ref):
    t = pl.program_id(1)
    wimg = sz_ref[0, 1]
    himg = sz_ref[0, 0]
    F = _decode(cand_ref[0], 0, CAP, wimg, himg)
    T = _decode(ctile_ref[0], t * RT, RT, wimg, himg)

    before = (F["score"][None, :] > T["score"][:, None]) | (
        (F["score"][None, :] == T["score"][:, None])
        & (F["ridx"][None, :] < T["ridx"][:, None]))
    rank = jnp.sum(before.astype(jnp.int32), axis=1).astype(jnp.float32)

    ltx = jnp.maximum(T["ox1"][:, None], F["ox1"][None, :])
    lty = jnp.maximum(T["oy1"][:, None], F["oy1"][None, :])
    rbx = jnp.minimum(T["ox2"][:, None], F["ox2"][None, :])
    rby = jnp.minimum(T["oy2"][:, None], F["oy2"][None, :])
    ww = jnp.clip(rbx - ltx, 0.0, None)
    hh = jnp.clip(rby - lty, 0.0, None)
    inter = ww * hh
    union = T["area"][:, None] + F["area"][None, :] - inter
    iou = inter / jnp.maximum(union, 1e-6)
    hot = ((iou > 0.6) & (F["det"][None, :] > T["det"][:, None])
           & F["valid"][None, :])
    sup = jnp.any(hot, axis=1)

    keep = jnp.where(T["valid"] & (~sup), T["det"], 0.0)
    out_ref[0] = jnp.stack(
        [keep, rank, T["x1"], T["y1"], T["x2"], T["y2"],
         T["cls"], keep], axis=0)


def _nms_b_body(mid_ref, out_ref):
    mid = mid_ref[0]
    keep = mid[0]
    rankf = mid[1]
    franks = []
    for t in range(CAP // RT):
        sl = slice(t * RT, (t + 1) * RT)
        b2 = (keep[None, :] > keep[sl][:, None]) | (
            (keep[None, :] == keep[sl][:, None])
            & (rankf[None, :] < rankf[sl][:, None]))
        franks.append(jnp.sum(b2.astype(jnp.int32), axis=1))
    frank = jnp.concatenate(franks)
    iota_r = lax.broadcasted_iota(jnp.int32, (128, CAP), 0)
    Q = (frank[None, :] == iota_r).astype(jnp.float32)
    G = jnp.stack([mid[2], mid[3], mid[4], mid[5], keep, mid[6],
                   keep, keep], axis=1)
    out_ref[0] = jnp.dot(Q, G, precision=lax.Precision.HIGHEST)


def _nms(imgsz_f32, cand):
    mid = pl.pallas_call(
        _nms_a_body,
        grid=(B, CAP // RT),
        in_specs=[
            pl.BlockSpec(memory_space=pltpu.SMEM),
            pl.BlockSpec((1, NF, CAP), lambda b, t: (b, 0, 0)),
            pl.BlockSpec((1, NF, RT), lambda b, t: (b, 0, t)),
        ],
        out_specs=pl.BlockSpec((1, 8, RT), lambda b, t: (b, 0, t)),
        out_shape=jax.ShapeDtypeStruct((B, 8, CAP), jnp.float32),
    )(imgsz_f32, cand, cand)
    return pl.pallas_call(
        _nms_b_body,
        grid=(B,),
        in_specs=[pl.BlockSpec((1, 8, CAP), lambda b: (b, 0, 0))],
        out_specs=pl.BlockSpec((1, 128, 8), lambda b: (b, 0, 0)),
        out_shape=jax.ShapeDtypeStruct((B, 128, 8), jnp.float32),
    )(mid)


def _excl_cumsum(x, axis):
    c = jnp.cumsum(x, axis=axis)
    return c - x


def _pick_threshold(counts, need):
    nb = counts.shape[1]
    cum = jnp.cumsum(counts[:, ::-1], axis=1)[:, ::-1]
    ge = cum >= need[:, None]
    idx = jnp.max(jnp.where(ge, jnp.arange(nb, dtype=jnp.int32)[None, :], -1),
                  axis=1)
    cum_pad = jnp.concatenate([cum, jnp.zeros((B, 1), jnp.int32)], axis=1)
    ngt = jnp.take_along_axis(cum_pad, (idx + 1)[:, None], axis=1)[:, 0]
    return idx, ngt


def _build_src(cnts, m):
    ngt = cnts[:, 0].reshape(B, TPB)
    neq = jnp.minimum(cnts[:, 1].reshape(B, TPB), CAP)
    gt_off = _excl_cumsum(ngt, 1)
    total_gt = jnp.sum(ngt, axis=1)
    eq_take = jnp.clip(m[:, None] - _excl_cumsum(neq, 1), 0, neq)
    eq_off = _excl_cumsum(eq_take, 1)
    j = jnp.arange(CAP, dtype=jnp.int32)[None, :]
    j2 = j - total_gt[:, None]
    bb = jnp.arange(B, dtype=jnp.int32)[:, None]
    src = jnp.zeros((B, CAP), jnp.int32)
    for t in range(TPB):
        go = gt_off[:, t:t + 1]
        in_g = (j >= go) & (j < go + ngt[:, t:t + 1])
        src = jnp.where(in_g, (bb * TPB + t) * CAP + (j - go), src)
        eo = eq_off[:, t:t + 1]
        in_e = (j2 >= eo) & (j2 < eo + eq_take[:, t:t + 1]) & (j < TOPN)
        src = jnp.where(in_e, NTILES * CAP + (bb * TPB + t) * CAP + (j2 - eo),
                        src)
    return src


def kernel(location, cls_pred, box_pred, center_pred, image_sizes):
    scores = _scores(cls_pred, center_pred)
    scores_flat = scores.reshape(B * N)

    need = jnp.full((B,), TOPN, jnp.int32)
    hist1 = _hist_hi()(scores_flat)
    h1 = jnp.sum(hist1.reshape(B, TPB, 16384), axis=1)
    bstar, ngt_hi = _pick_threshold(h1, need)

    bstar16 = jnp.zeros((16,), jnp.int32).at[:B].set(bstar)
    hist2 = _hist_lo()(scores_flat, bstar16)
    h2 = jnp.sum(hist2.reshape(B, TPB, 65536), axis=1)
    vlo, ngt_lo = _pick_threshold(h2, need - ngt_hi)
    vstar = jnp.left_shift(bstar, 16) | vlo
    n_gt = ngt_hi + ngt_lo
    m = need - n_gt

    vstar16 = jnp.zeros((16,), jnp.int32).at[:B].set(vstar)
    gt_idx, eq_idx, cnts = _collect()(scores_flat, vstar16)
    src = _build_src(cnts, m)

    lists_cat = jnp.concatenate([gt_idx.reshape(-1), eq_idx.reshape(-1)])
    cand = _gather()(lists_cat, src.reshape(-1), scores_flat,
                     location.reshape(-1), box_pred.reshape(-1))

    imgsz = image_sizes.astype(jnp.float32).reshape(1, 2)
    raw = _nms(imgsz, cand)
    detections = raw[:, :100, :5]
    labels = raw[:, :100, 5].astype(jnp.int32)
    return detections, labels


# speedup vs baseline: 5.1941x; 1.1624x over previous
import functools
import jax
import jax.numpy as jnp
from jax import lax
from jax.experimental import pallas as pl
from jax.experimental.pallas import tpu as pltpu
from jax.experimental.pallas import tpu_sc as plsc

H = 100
W = 200
HW = H * W
C = 80
B = 8
N = HW * C
TOPN = 1000
NTILES = 32
TPB = NTILES // B
RNG = N // TPB
CHUNK = 4000
NCHUNK = RNG // CHUNK
UNROLL = 10
CAP = 1024
PSC = 2000

@functools.lru_cache(maxsize=None)
def _mesh():
    return plsc.VectorSubcoreMesh(core_axis_name="c", subcore_axis_name="s")


def _wid():
    return lax.axis_index("s") * 2 + lax.axis_index("c")


def _score_body(cls_ref, ctr_ref, out_ref):
    cls = jax.nn.sigmoid(cls_ref[0].reshape(C, HW))
    ctr = jax.nn.sigmoid(ctr_ref[0].reshape(1, HW))
    s = jnp.where(cls > 0.05, cls * ctr, 0.0)
    out_ref[0] = s.T


def _scores(cls_pred, center_pred):
    return pl.pallas_call(
        _score_body,
        grid=(B,),
        in_specs=[
            pl.BlockSpec((1, C, H, W), lambda b: (b, 0, 0, 0)),
            pl.BlockSpec((1, 1, H, W), lambda b: (b, 0, 0, 0)),
        ],
        out_specs=pl.BlockSpec((1, HW, C), lambda b: (b, 0, 0)),
        out_shape=jax.ShapeDtypeStruct((B, HW, C), jnp.float32),
    )(cls_pred, center_pred)


def _hist_hi_body(scores_hbm, hist_out, chunk_v, hist_v, sem):
    wid = _wid()
    base = wid * RNG

    def zero(i, _):
        hist_v[pl.ds(i * 16, 16)] = jnp.zeros((16,), jnp.int32)
        return 0
    lax.fori_loop(0, 16384 // 16, zero, 0)

    ones = jnp.ones((16,), jnp.int32)

    def chunk(k, _):
        pltpu.sync_copy(scores_hbm.at[pl.ds(base + k * CHUNK, CHUNK)], chunk_v)

        def inner(i, _):
            for u in range(UNROLL):
                v = chunk_v[pl.ds((i * UNROLL + u) * 16, 16)]
                bits = lax.bitcast_convert_type(v, jnp.int32)
                hi = lax.shift_right_logical(bits, 16)
                plsc.addupdate_scatter(hist_v, [hi], ones, mask=bits >= 0)
            return 0
        lax.fori_loop(0, CHUNK // 16 // UNROLL, inner, 0)
        return 0
    lax.fori_loop(0, NCHUNK, chunk, 0)
    pltpu.sync_copy(hist_v, hist_out.at[wid])


@functools.lru_cache(maxsize=None)
def _hist_hi():
    return functools.partial(
        pl.kernel, mesh=_mesh(),
        compiler_params=pltpu.CompilerParams(
            needs_layout_passes=False, use_tc_tiling_on_sc=False),
        out_type=jax.ShapeDtypeStruct((NTILES, 16384), jnp.int32),
        scratch_types=[
            pltpu.VMEM((CHUNK,), jnp.float32),
            pltpu.VMEM((16384,), jnp.int32),
            pltpu.SemaphoreType.DMA,
        ],
    )(_hist_hi_body)


def _sel_lane(vec16, lane):
    msk = lax.iota(jnp.int32, 16) == lane
    return jnp.max(jnp.where(msk, vec16, jnp.int32(-2147483648)))


def _hist_lo_body(scores_hbm, bstar_hbm, hist_out, chunk_v, hist_v, bst_v, sem):
    wid = _wid()
    base = wid * RNG
    b = wid // TPB
    pltpu.sync_copy(bstar_hbm, bst_v)
    bstar = _sel_lane(bst_v[...], b)

    def zero(i, _):
        hist_v[pl.ds(i * 16, 16)] = jnp.zeros((16,), jnp.int32)
        return 0
    lax.fori_loop(0, 65536 // 16, zero, 0)

    ones = jnp.ones((16,), jnp.int32)

    def chunk(k, _):
        pltpu.sync_copy(scores_hbm.at[pl.ds(base + k * CHUNK, CHUNK)], chunk_v)

        def inner(i, _):
            acc = None
            for u in range(UNROLL):
                v = chunk_v[pl.ds((i * UNROLL + u) * 16, 16)]
                bits = lax.bitcast_convert_type(v, jnp.int32)
                m = lax.shift_right_logical(bits, 16) == bstar
                acc = m if acc is None else (acc | m)

            @pl.when(jnp.any(acc))
            def _():
                for u in range(UNROLL):
                    v = chunk_v[pl.ds((i * UNROLL + u) * 16, 16)]
                    bits = lax.bitcast_convert_type(v, jnp.int32)
                    hi = lax.shift_right_logical(bits, 16)
                    lo = jnp.bitwise_and(bits, jnp.int32(0xFFFF))
                    plsc.addupdate_scatter(hist_v, [lo], ones,
                                           mask=hi == bstar)
            return 0
        lax.fori_loop(0, CHUNK // 16 // UNROLL, inner, 0)
        return 0
    lax.fori_loop(0, NCHUNK, chunk, 0)
    pltpu.sync_copy(hist_v, hist_out.at[wid])


@functools.lru_cache(maxsize=None)
def _hist_lo():
    return functools.partial(
        pl.kernel, mesh=_mesh(),
        compiler_params=pltpu.CompilerParams(
            needs_layout_passes=False, use_tc_tiling_on_sc=False),
        out_type=jax.ShapeDtypeStruct((NTILES, 65536), jnp.int32),
        scratch_types=[
            pltpu.VMEM((CHUNK,), jnp.float32),
            pltpu.VMEM((65536,), jnp.int32),
            pltpu.VMEM((16,), jnp.int32),
            pltpu.SemaphoreType.DMA,
        ],
    )(_hist_lo_body)


def _collect_body(scores_hbm, vstar_hbm, gt_out, eq_out, cnt_out,
                  chunk_v, gt_v, eq_v, vst_v, cnt_v, cntg_v, cnte_v, sem):
    wid = _wid()
    base = wid * RNG
    b = wid // TPB
    ibase = (wid % TPB) * RNG
    pltpu.sync_copy(vstar_hbm, vst_v)
    vstar = _sel_lane(vst_v[...], b)
    lanes = lax.iota(jnp.int32, 16)
    cntg_v[...] = jnp.zeros((16,), jnp.int32)
    cnte_v[...] = jnp.zeros((16,), jnp.int32)

    def zero(i, _):
        gt_v[pl.ds(i * 16, 16)] = jnp.zeros((16,), jnp.int32)
        eq_v[pl.ds(i * 16, 16)] = jnp.zeros((16,), jnp.int32)
        return 0
    lax.fori_loop(0, CAP // 16, zero, 0)

    def chunk(k, _):
        pltpu.sync_copy(scores_hbm.at[pl.ds(base + k * CHUNK, CHUNK)], chunk_v)

        def inner(i, _):
            acc = None
            for u in range(UNROLL):
                v = chunk_v[pl.ds((i * UNROLL + u) * 16, 16)]
                bits = lax.bitcast_convert_type(v, jnp.int32)
                m = bits >= vstar
                acc = m if acc is None else (acc | m)

            @pl.when(jnp.any(acc))
            def _():
                cg = cntg_v[...]
                ce = cnte_v[...]
                for u in range(UNROLL):
                    v = chunk_v[pl.ds((i * UNROLL + u) * 16, 16)]
                    bits = lax.bitcast_convert_type(v, jnp.int32)
                    gidx = (ibase + k * CHUNK + (i * UNROLL + u) * 16) + lanes
                    gm = bits > vstar
                    em = bits == vstar
                    gpos = cg + plsc.cumsum(gm.astype(jnp.int32)) - 1
                    plsc.store_scatter(gt_v, [gpos], gidx,
                                       mask=gm & (gpos < CAP))
                    epos = ce + plsc.cumsum(em.astype(jnp.int32)) - 1
                    plsc.store_scatter(eq_v, [epos], gidx,
                                       mask=em & (epos < CAP))
                    cg = cg + jnp.sum(gm.astype(jnp.int32))
                    ce = ce + jnp.sum(em.astype(jnp.int32))
                cntg_v[...] = cg
                cnte_v[...] = ce
            return 0
        lax.fori_loop(0, CHUNK // 16 // UNROLL, inner, 0)
        return 0

    lax.fori_loop(0, NCHUNK, chunk, 0)
    cgt = jnp.max(cntg_v[...])
    ceq = jnp.max(cnte_v[...])
    cnt_v[...] = jnp.where(lanes == 0, cgt, jnp.where(lanes == 1, ceq, 0))
    pltpu.sync_copy(gt_v, gt_out.at[wid])
    pltpu.sync_copy(eq_v, eq_out.at[wid])
    pltpu.sync_copy(cnt_v, cnt_out.at[wid])


@functools.lru_cache(maxsize=None)
def _collect():
    return functools.partial(
        pl.kernel, mesh=_mesh(),
        compiler_params=pltpu.CompilerParams(
            needs_layout_passes=False, use_tc_tiling_on_sc=False),
        out_type=(
            jax.ShapeDtypeStruct((NTILES, CAP), jnp.int32),
            jax.ShapeDtypeStruct((NTILES, CAP), jnp.int32),
            jax.ShapeDtypeStruct((NTILES, 16), jnp.int32),
        ),
        scratch_types=[
            pltpu.VMEM((CHUNK,), jnp.float32),
            pltpu.VMEM((CAP,), jnp.int32),
            pltpu.VMEM((CAP,), jnp.int32),
            pltpu.VMEM((16,), jnp.int32),
            pltpu.VMEM((16,), jnp.int32),
            pltpu.VMEM((16,), jnp.int32),
            pltpu.VMEM((16,), jnp.int32),
            pltpu.SemaphoreType.DMA,
        ],
    )(_collect_body)


SLOTS = CAP // TPB
NF = 16


def _gather_body(lists_hbm, src_hbm, scores_hbm, loc_hbm, box_hbm, out_hbm,
                 src_v, cand_v, addr_v, fld_v, sem):
    wid = _wid()
    b = wid // TPB
    q = wid % TPB
    pltpu.sync_copy(src_hbm.at[pl.ds(b * CAP + q * SLOTS, SLOTS)], src_v)
    pltpu.async_copy(lists_hbm.at[src_v], cand_v, sem).wait()

    def addrs(i, field, fn):
        def body(j, _):
            cand = cand_v[pl.ds(j * 16, 16)]
            addr_v[pl.ds(j * 16, 16)] = fn(cand)
            return 0
        lax.fori_loop(0, SLOTS // 16, body, 0)

    addrs(0, 0, lambda cand: b * N + cand)
    pltpu.async_copy(scores_hbm.at[addr_v], fld_v.at[0], sem).wait()
    addrs(0, 1, lambda cand: 2 * (cand // C))
    pltpu.async_copy(loc_hbm.at[addr_v], fld_v.at[1], sem).wait()
    addrs(0, 2, lambda cand: 2 * (cand // C) + 1)
    pltpu.async_copy(loc_hbm.at[addr_v], fld_v.at[2], sem).wait()
    for f in range(4):
        addrs(0, 3 + f, lambda cand, f=f: (b * 4 + f) * HW + cand // C)
        pltpu.async_copy(box_hbm.at[addr_v], fld_v.at[3 + f], sem).wait()

    def cls_body(j, _):
        cand = cand_v[pl.ds(j * 16, 16)]
        c = cand - (cand // C) * C
        fld_v[7, pl.ds(j * 16, 16)] = (c + 1).astype(jnp.float32)
        fld_v[8, pl.ds(j * 16, 16)] = cand.astype(jnp.float32)
        return 0
    lax.fori_loop(0, SLOTS // 16, cls_body, 0)

    for f in range(9):
        pltpu.sync_copy(fld_v.at[f], out_hbm.at[b, f, pl.ds(q * SLOTS, SLOTS)])


@functools.lru_cache(maxsize=None)
def _gather():
    return functools.partial(
        pl.kernel, mesh=_mesh(),
        compiler_params=pltpu.CompilerParams(
            needs_layout_passes=False, use_tc_tiling_on_sc=False),
        out_type=jax.ShapeDtypeStruct((B, NF, CAP), jnp.float32),
        scratch_types=[
            pltpu.VMEM((SLOTS,), jnp.int32),
            pltpu.VMEM((SLOTS,), jnp.int32),
            pltpu.VMEM((SLOTS,), jnp.int32),
            pltpu.VMEM((NF, SLOTS), jnp.float32),
            pltpu.SemaphoreType.DMA,
        ],
    )(_gather_body)


RT = 128


def _decode(cand, jbase, nlanes, wimg, himg):
    j_i32 = jbase + lax.broadcasted_iota(jnp.int32, (nlanes,), 0)
    jf = j_i32.astype(jnp.float32)
    padm = j_i32 >= TOPN
    score = jnp.where(padm, -1.0, cand[0])
    ridx = jnp.where(padm, 2.0e6 + jf, cand[8])
    s = jnp.where(padm, 0.0, cand[0])
    x1 = jnp.clip(cand[1] - cand[3], 0.0, wimg - 1.0)
    y1 = jnp.clip(cand[2] - cand[4], 0.0, himg - 1.0)
    x2 = jnp.clip(cand[1] + cand[5], 0.0, wimg - 1.0)
    y2 = jnp.clip(cand[2] + cand[6], 0.0, himg - 1.0)
    det = jnp.sqrt(jnp.maximum(s, 1e-12))
    valid = s > 0.0
    off = cand[7] * (jnp.maximum(wimg, himg) + 1.0)
    area = jnp.maximum(x2 - x1, 0.0) * jnp.maximum(y2 - y1, 0.0)
    return dict(score=score, ridx=ridx, det=det, valid=valid, x1=x1, y1=y1,
                x2=x2, y2=y2, ox1=x1 + off, oy1=y1 + off, ox2=x2 + off,
                oy2=y2 + off, area=area, cls=cand[7])


def _nms_a_body(sz_ref, cand_ref, ctile_ref, out_ref):
    t = pl.program_id(1)
    wimg = sz_ref[0, 1]
    himg = sz_ref[0, 0]
    F = _decode(cand_ref[0], 0, CAP, wimg, himg)
    T = _decode(ctile_ref[0], t * RT, RT, wimg, himg)

    before = (F["score"][None, :] > T["score"][:, None]) | (
        (F["score"][None, :] == T["score"][:, None])
        & (F["ridx"][None, :] < T["ridx"][:, None]))
    rank = jnp.sum(before.astype(jnp.int32), axis=1).astype(jnp.float32)

    ltx = jnp.maximum(T["ox1"][:, None], F["ox1"][None, :])
    lty = jnp.maximum(T["oy1"][:, None], F["oy1"][None, :])
    rbx = jnp.minimum(T["ox2"][:, None], F["ox2"][None, :])
    rby = jnp.minimum(T["oy2"][:, None], F["oy2"][None, :])
    ww = jnp.clip(rbx - ltx, 0.0, None)
    hh = jnp.clip(rby - lty, 0.0, None)
    inter = ww * hh
    union = T["area"][:, None] + F["area"][None, :] - inter
    iou = inter / jnp.maximum(union, 1e-6)
    hot = ((iou > 0.6) & (F["det"][None, :] > T["det"][:, None])
           & F["valid"][None, :])
    sup = jnp.any(hot, axis=1)

    keep = jnp.where(T["valid"] & (~sup), T["det"], 0.0)
    out_ref[0] = jnp.stack(
        [keep, rank, T["x1"], T["y1"], T["x2"], T["y2"],
         T["cls"], keep], axis=0)


def _nms_b_body(mid_ref, out_ref):
    mid = mid_ref[0]
    keep = mid[0]
    rankf = mid[1]
    franks = []
    for t in range(CAP // RT):
        sl = slice(t * RT, (t + 1) * RT)
        b2 = (keep[None, :] > keep[sl][:, None]) | (
            (keep[None, :] == keep[sl][:, None])
            & (rankf[None, :] < rankf[sl][:, None]))
        franks.append(jnp.sum(b2.astype(jnp.int32), axis=1))
    frank = jnp.concatenate(franks)
    iota_r = lax.broadcasted_iota(jnp.int32, (128, CAP), 0)
    Q = (frank[None, :] == iota_r).astype(jnp.float32)
    G = jnp.stack([mid[2], mid[3], mid[4], mid[5], keep, mid[6],
                   keep, keep], axis=1)
    out_ref[0] = jnp.dot(Q, G, precision=lax.Precision.HIGHEST)


def _nms(imgsz_f32, cand):
    mid = pl.pallas_call(
        _nms_a_body,
        grid=(B, CAP // RT),
        in_specs=[
            pl.BlockSpec(memory_space=pltpu.SMEM),
            pl.BlockSpec((1, NF, CAP), lambda b, t: (b, 0, 0)),
            pl.BlockSpec((1, NF, RT), lambda b, t: (b, 0, t)),
        ],
        out_specs=pl.BlockSpec((1, 8, RT), lambda b, t: (b, 0, t)),
        out_shape=jax.ShapeDtypeStruct((B, 8, CAP), jnp.float32),
    )(imgsz_f32, cand, cand)
    return pl.pallas_call(
        _nms_b_body,
        grid=(B,),
        in_specs=[pl.BlockSpec((1, 8, CAP), lambda b: (b, 0, 0))],
        out_specs=pl.BlockSpec((1, 128, 8), lambda b: (b, 0, 0)),
        out_shape=jax.ShapeDtypeStruct((B, 128, 8), jnp.float32),
    )(mid)


def _excl_cumsum(x, axis):
    c = jnp.cumsum(x, axis=axis)
    return c - x


def _pick_threshold(counts, need):
    nb = counts.shape[1]
    cum = jnp.cumsum(counts[:, ::-1], axis=1)[:, ::-1]
    ge = cum >= need[:, None]
    idx = jnp.max(jnp.where(ge, jnp.arange(nb, dtype=jnp.int32)[None, :], -1),
                  axis=1)
    cum_pad = jnp.concatenate([cum, jnp.zeros((B, 1), jnp.int32)], axis=1)
    ngt = jnp.take_along_axis(cum_pad, (idx + 1)[:, None], axis=1)[:, 0]
    return idx, ngt


def _build_src(cnts, m):
    ngt = cnts[:, 0].reshape(B, TPB)
    neq = jnp.minimum(cnts[:, 1].reshape(B, TPB), CAP)
    gt_off = _excl_cumsum(ngt, 1)
    total_gt = jnp.sum(ngt, axis=1)
    eq_take = jnp.clip(m[:, None] - _excl_cumsum(neq, 1), 0, neq)
    eq_off = _excl_cumsum(eq_take, 1)
    j = jnp.arange(CAP, dtype=jnp.int32)[None, :]
    j2 = j - total_gt[:, None]
    bb = jnp.arange(B, dtype=jnp.int32)[:, None]
    src = jnp.zeros((B, CAP), jnp.int32)
    for t in range(TPB):
        go = gt_off[:, t:t + 1]
        in_g = (j >= go) & (j < go + ngt[:, t:t + 1])
        src = jnp.where(in_g, (bb * TPB + t) * CAP + (j - go), src)
        eo = eq_off[:, t:t + 1]
        in_e = (j2 >= eo) & (j2 < eo + eq_take[:, t:t + 1]) & (j < TOPN)
        src = jnp.where(in_e, NTILES * CAP + (bb * TPB + t) * CAP + (j2 - eo),
                        src)
    return src


def kernel(location, cls_pred, box_pred, center_pred, image_sizes):
    scores = _scores(cls_pred, center_pred)
    scores_flat = scores.reshape(B * N)

    need = jnp.full((B,), TOPN, jnp.int32)
    hist1 = _hist_hi()(scores_flat)
    h1 = jnp.sum(hist1.reshape(B, TPB, 16384), axis=1)
    bstar, ngt_hi = _pick_threshold(h1, need)

    bstar16 = jnp.zeros((16,), jnp.int32).at[:B].set(bstar)
    hist2 = _hist_lo()(scores_flat, bstar16)
    h2 = jnp.sum(hist2.reshape(B, TPB, 65536), axis=1)
    vlo, ngt_lo = _pick_threshold(h2, need - ngt_hi)
    vstar = jnp.left_shift(bstar, 16) | vlo
    n_gt = ngt_hi + ngt_lo
    m = need - n_gt

    vstar16 = jnp.zeros((16,), jnp.int32).at[:B].set(vstar)
    gt_idx, eq_idx, cnts = _collect()(scores_flat, vstar16)
    src = _build_src(cnts, m)

    lists_cat = jnp.concatenate([gt_idx.reshape(-1), eq_idx.reshape(-1)])
    cand = _gather()(lists_cat, src.reshape(-1), scores_flat,
                     location.reshape(-1), box_pred.reshape(-1))

    imgsz = image_sizes.astype(jnp.float32).reshape(1, 2)
    raw = _nms(imgsz, cand)
    detections = raw[:, :100, :5]
    labels = raw[:, :100, 5].astype(jnp.int32)
    return detections, labels


# speedup vs baseline: 5.4027x; 1.0402x over previous
import functools
import jax
import jax.numpy as jnp
from jax import lax
from jax.experimental import pallas as pl
from jax.experimental.pallas import tpu as pltpu
from jax.experimental.pallas import tpu_sc as plsc

H = 100
W = 200
HW = H * W
C = 80
B = 8
N = HW * C
TOPN = 1000
NTILES = 32
TPB = NTILES // B
RNG = N // TPB
CHUNK = 4000
NCHUNK = RNG // CHUNK
UNROLL = 10
CAP = 1024
PSC = 2000

@functools.lru_cache(maxsize=None)
def _mesh():
    return plsc.VectorSubcoreMesh(core_axis_name="c", subcore_axis_name="s")


def _wid():
    return lax.axis_index("s") * 2 + lax.axis_index("c")


def _score_body(cls_ref, ctr_ref, out_ref):
    cls = jax.nn.sigmoid(cls_ref[0].reshape(C, HW))
    ctr = jax.nn.sigmoid(ctr_ref[0].reshape(1, HW))
    s = jnp.where(cls > 0.05, cls * ctr, 0.0)
    out_ref[0] = s.T


def _scores(cls_pred, center_pred):
    return pl.pallas_call(
        _score_body,
        grid=(B,),
        in_specs=[
            pl.BlockSpec((1, C, H, W), lambda b: (b, 0, 0, 0)),
            pl.BlockSpec((1, 1, H, W), lambda b: (b, 0, 0, 0)),
        ],
        out_specs=pl.BlockSpec((1, HW, C), lambda b: (b, 0, 0)),
        out_shape=jax.ShapeDtypeStruct((B, HW, C), jnp.float32),
    )(cls_pred, center_pred)


def _hist_hi_body(scores_hbm, hist_out, chunk_v, hist_v, histb_v, sem):
    wid = _wid()
    base = wid * RNG

    def zero(i, _):
        hist_v[pl.ds(i * 16, 16)] = jnp.zeros((16,), jnp.int32)
        histb_v[pl.ds(i * 16, 16)] = jnp.zeros((16,), jnp.int32)
        return 0
    lax.fori_loop(0, 16384 // 16, zero, 0)

    ones = jnp.ones((16,), jnp.int32)

    def chunk(k, _):
        pltpu.sync_copy(scores_hbm.at[pl.ds(base + k * CHUNK, CHUNK)], chunk_v)

        def inner(i, _):
            for u in range(UNROLL):
                v = chunk_v[pl.ds((i * UNROLL + u) * 16, 16)]
                bits = lax.bitcast_convert_type(v, jnp.int32)
                hi = lax.shift_right_logical(bits, 16)
                tgt = hist_v if u % 2 == 0 else histb_v
                plsc.addupdate_scatter(tgt, [hi], ones, mask=bits >= 0)
            return 0
        lax.fori_loop(0, CHUNK // 16 // UNROLL, inner, 0)
        return 0
    lax.fori_loop(0, NCHUNK, chunk, 0)

    def merge(i, _):
        sl = pl.ds(i * 16, 16)
        hist_v[sl] = hist_v[sl] + histb_v[sl]
        return 0
    lax.fori_loop(0, 16384 // 16, merge, 0)
    pltpu.sync_copy(hist_v, hist_out.at[wid])


@functools.lru_cache(maxsize=None)
def _hist_hi():
    return functools.partial(
        pl.kernel, mesh=_mesh(),
        compiler_params=pltpu.CompilerParams(
            needs_layout_passes=False, use_tc_tiling_on_sc=False),
        out_type=jax.ShapeDtypeStruct((NTILES, 16384), jnp.int32),
        scratch_types=[
            pltpu.VMEM((CHUNK,), jnp.float32),
            pltpu.VMEM((16384,), jnp.int32),
            pltpu.VMEM((16384,), jnp.int32),
            pltpu.SemaphoreType.DMA,
        ],
    )(_hist_hi_body)


def _sel_lane(vec16, lane):
    msk = lax.iota(jnp.int32, 16) == lane
    return jnp.max(jnp.where(msk, vec16, jnp.int32(-2147483648)))


def _hist_lo_body(scores_hbm, bstar_hbm, hist_out, chunk_v, hist_v, bst_v, sem):
    wid = _wid()
    base = wid * RNG
    b = wid // TPB
    pltpu.sync_copy(bstar_hbm, bst_v)
    bstar = _sel_lane(bst_v[...], b)

    def zero(i, _):
        hist_v[pl.ds(i * 16, 16)] = jnp.zeros((16,), jnp.int32)
        return 0
    lax.fori_loop(0, 65536 // 16, zero, 0)

    ones = jnp.ones((16,), jnp.int32)

    def chunk(k, _):
        pltpu.sync_copy(scores_hbm.at[pl.ds(base + k * CHUNK, CHUNK)], chunk_v)

        def inner(i, _):
            for u in range(UNROLL):
                v = chunk_v[pl.ds((i * UNROLL + u) * 16, 16)]
                bits = lax.bitcast_convert_type(v, jnp.int32)
                hi = lax.shift_right_logical(bits, 16)
                lo = jnp.bitwise_and(bits, jnp.int32(0xFFFF))
                plsc.addupdate_scatter(hist_v, [lo], ones, mask=hi == bstar)
            return 0
        lax.fori_loop(0, CHUNK // 16 // UNROLL, inner, 0)
        return 0
    lax.fori_loop(0, NCHUNK, chunk, 0)
    pltpu.sync_copy(hist_v, hist_out.at[wid])


@functools.lru_cache(maxsize=None)
def _hist_lo():
    return functools.partial(
        pl.kernel, mesh=_mesh(),
        compiler_params=pltpu.CompilerParams(
            needs_layout_passes=False, use_tc_tiling_on_sc=False),
        out_type=jax.ShapeDtypeStruct((NTILES, 65536), jnp.int32),
        scratch_types=[
            pltpu.VMEM((CHUNK,), jnp.float32),
            pltpu.VMEM((65536,), jnp.int32),
            pltpu.VMEM((16,), jnp.int32),
            pltpu.SemaphoreType.DMA,
        ],
    )(_hist_lo_body)


def _collect_body(scores_hbm, vstar_hbm, gt_out, eq_out, cnt_out,
                  chunk_v, gt_v, eq_v, vst_v, cnt_v, cntg_v, cnte_v, sem):
    wid = _wid()
    base = wid * RNG
    b = wid // TPB
    ibase = (wid % TPB) * RNG
    pltpu.sync_copy(vstar_hbm, vst_v)
    vstar = _sel_lane(vst_v[...], b)
    lanes = lax.iota(jnp.int32, 16)
    cntg_v[...] = jnp.zeros((16,), jnp.int32)
    cnte_v[...] = jnp.zeros((16,), jnp.int32)

    def zero(i, _):
        gt_v[pl.ds(i * 16, 16)] = jnp.zeros((16,), jnp.int32)
        eq_v[pl.ds(i * 16, 16)] = jnp.zeros((16,), jnp.int32)
        return 0
    lax.fori_loop(0, CAP // 16, zero, 0)

    def chunk(k, _):
        pltpu.sync_copy(scores_hbm.at[pl.ds(base + k * CHUNK, CHUNK)], chunk_v)

        def inner(i, _):
            acc = None
            for u in range(UNROLL):
                v = chunk_v[pl.ds((i * UNROLL + u) * 16, 16)]
                bits = lax.bitcast_convert_type(v, jnp.int32)
                m = bits >= vstar
                acc = m if acc is None else (acc | m)

            @pl.when(jnp.any(acc))
            def _():
                cg = cntg_v[...]
                ce = cnte_v[...]
                for u in range(UNROLL):
                    v = chunk_v[pl.ds((i * UNROLL + u) * 16, 16)]
                    bits = lax.bitcast_convert_type(v, jnp.int32)
                    gidx = (ibase + k * CHUNK + (i * UNROLL + u) * 16) + lanes
                    gm = bits > vstar
                    em = bits == vstar
                    gpos = cg + plsc.cumsum(gm.astype(jnp.int32)) - 1
                    plsc.store_scatter(gt_v, [gpos], gidx,
                                       mask=gm & (gpos < CAP))
                    epos = ce + plsc.cumsum(em.astype(jnp.int32)) - 1
                    plsc.store_scatter(eq_v, [epos], gidx,
                                       mask=em & (epos < CAP))
                    cg = cg + jnp.sum(gm.astype(jnp.int32))
                    ce = ce + jnp.sum(em.astype(jnp.int32))
                cntg_v[...] = cg
                cnte_v[...] = ce
            return 0
        lax.fori_loop(0, CHUNK // 16 // UNROLL, inner, 0)
        return 0

    lax.fori_loop(0, NCHUNK, chunk, 0)
    cgt = jnp.max(cntg_v[...])
    ceq = jnp.max(cnte_v[...])
    cnt_v[...] = jnp.where(lanes == 0, cgt, jnp.where(lanes == 1, ceq, 0))
    pltpu.sync_copy(gt_v, gt_out.at[wid])
    pltpu.sync_copy(eq_v, eq_out.at[wid])
    pltpu.sync_copy(cnt_v, cnt_out.at[wid])


@functools.lru_cache(maxsize=None)
def _collect():
    return functools.partial(
        pl.kernel, mesh=_mesh(),
        compiler_params=pltpu.CompilerParams(
            needs_layout_passes=False, use_tc_tiling_on_sc=False),
        out_type=(
            jax.ShapeDtypeStruct((NTILES, CAP), jnp.int32),
            jax.ShapeDtypeStruct((NTILES, CAP), jnp.int32),
            jax.ShapeDtypeStruct((NTILES, 16), jnp.int32),
        ),
        scratch_types=[
            pltpu.VMEM((CHUNK,), jnp.float32),
            pltpu.VMEM((CAP,), jnp.int32),
            pltpu.VMEM((CAP,), jnp.int32),
            pltpu.VMEM((16,), jnp.int32),
            pltpu.VMEM((16,), jnp.int32),
            pltpu.VMEM((16,), jnp.int32),
            pltpu.VMEM((16,), jnp.int32),
            pltpu.SemaphoreType.DMA,
        ],
    )(_collect_body)


SLOTS = CAP // TPB
NF = 16


def _gather_body(lists_hbm, src_hbm, scores_hbm, loc_hbm, box_hbm, out_hbm,
                 src_v, cand_v, addr_v, fld_v, sem):
    wid = _wid()
    b = wid // TPB
    q = wid % TPB
    pltpu.sync_copy(src_hbm.at[pl.ds(b * CAP + q * SLOTS, SLOTS)], src_v)
    pltpu.async_copy(lists_hbm.at[src_v], cand_v, sem).wait()

    def addrs(i, field, fn):
        def body(j, _):
            cand = cand_v[pl.ds(j * 16, 16)]
            addr_v[pl.ds(j * 16, 16)] = fn(cand)
            return 0
        lax.fori_loop(0, SLOTS // 16, body, 0)

    addrs(0, 0, lambda cand: b * N + cand)
    pltpu.async_copy(scores_hbm.at[addr_v], fld_v.at[0], sem).wait()
    addrs(0, 1, lambda cand: 2 * (cand // C))
    pltpu.async_copy(loc_hbm.at[addr_v], fld_v.at[1], sem).wait()
    addrs(0, 2, lambda cand: 2 * (cand // C) + 1)
    pltpu.async_copy(loc_hbm.at[addr_v], fld_v.at[2], sem).wait()
    for f in range(4):
        addrs(0, 3 + f, lambda cand, f=f: (b * 4 + f) * HW + cand // C)
        pltpu.async_copy(box_hbm.at[addr_v], fld_v.at[3 + f], sem).wait()

    def cls_body(j, _):
        cand = cand_v[pl.ds(j * 16, 16)]
        c = cand - (cand // C) * C
        fld_v[7, pl.ds(j * 16, 16)] = (c + 1).astype(jnp.float32)
        fld_v[8, pl.ds(j * 16, 16)] = cand.astype(jnp.float32)
        return 0
    lax.fori_loop(0, SLOTS // 16, cls_body, 0)

    for f in range(9):
        pltpu.sync_copy(fld_v.at[f], out_hbm.at[b, f, pl.ds(q * SLOTS, SLOTS)])


@functools.lru_cache(maxsize=None)
def _gather():
    return functools.partial(
        pl.kernel, mesh=_mesh(),
        compiler_params=pltpu.CompilerParams(
            needs_layout_passes=False, use_tc_tiling_on_sc=False),
        out_type=jax.ShapeDtypeStruct((B, NF, CAP), jnp.float32),
        scratch_types=[
            pltpu.VMEM((SLOTS,), jnp.int32),
            pltpu.VMEM((SLOTS,), jnp.int32),
            pltpu.VMEM((SLOTS,), jnp.int32),
            pltpu.VMEM((NF, SLOTS), jnp.float32),
            pltpu.SemaphoreType.DMA,
        ],
    )(_gather_body)


RT = 128


def _decode(cand, jbase, nlanes, wimg, himg):
    j_i32 = jbase + lax.broadcasted_iota(jnp.int32, (nlanes,), 0)
    jf = j_i32.astype(jnp.float32)
    padm = j_i32 >= TOPN
    score = jnp.where(padm, -1.0, cand[0])
    ridx = jnp.where(padm, 2.0e6 + jf, cand[8])
    s = jnp.where(padm, 0.0, cand[0])
    x1 = jnp.clip(cand[1] - cand[3], 0.0, wimg - 1.0)
    y1 = jnp.clip(cand[2] - cand[4], 0.0, himg - 1.0)
    x2 = jnp.clip(cand[1] + cand[5], 0.0, wimg - 1.0)
    y2 = jnp.clip(cand[2] + cand[6], 0.0, himg - 1.0)
    det = jnp.sqrt(jnp.maximum(s, 1e-12))
    valid = s > 0.0
    off = cand[7] * (jnp.maximum(wimg, himg) + 1.0)
    area = jnp.maximum(x2 - x1, 0.0) * jnp.maximum(y2 - y1, 0.0)
    return dict(score=score, ridx=ridx, det=det, valid=valid, x1=x1, y1=y1,
                x2=x2, y2=y2, ox1=x1 + off, oy1=y1 + off, ox2=x2 + off,
                oy2=y2 + off, area=area, cls=cand[7])


def _nms_a_body(sz_ref, cand_ref, ctile_ref, out_ref):
    t = pl.program_id(1)
    wimg = sz_ref[0, 1]
    himg = sz_ref[0, 0]
    F = _decode(cand_ref[0], 0, CAP, wimg, himg)
    T = _decode(ctile_ref[0], t * RT, RT, wimg, himg)

    before = (F["score"][None, :] > T["score"][:, None]) | (
        (F["score"][None, :] == T["score"][:, None])
        & (F["ridx"][None, :] < T["ridx"][:, None]))
    rank = jnp.sum(before.astype(jnp.int32), axis=1).astype(jnp.float32)

    ltx = jnp.maximum(T["ox1"][:, None], F["ox1"][None, :])
    lty = jnp.maximum(T["oy1"][:, None], F["oy1"][None, :])
    rbx = jnp.minimum(T["ox2"][:, None], F["ox2"][None, :])
    rby = jnp.minimum(T["oy2"][:, None], F["oy2"][None, :])
    ww = jnp.clip(rbx - ltx, 0.0, None)
    hh = jnp.clip(rby - lty, 0.0, None)
    inter = ww * hh
    union = T["area"][:, None] + F["area"][None, :] - inter
    iou = inter / jnp.maximum(union, 1e-6)
    hot = ((iou > 0.6) & (F["det"][None, :] > T["det"][:, None])
           & F["valid"][None, :])
    sup = jnp.any(hot, axis=1)

    keep = jnp.where(T["valid"] & (~sup), T["det"], 0.0)
    out_ref[0] = jnp.stack(
        [keep, rank, T["x1"], T["y1"], T["x2"], T["y2"],
         T["cls"], keep], axis=0)


def _nms_b_body(mid_ref, out_ref):
    mid = mid_ref[0]
    keep = mid[0]
    rankf = mid[1]
    franks = []
    for t in range(CAP // RT):
        sl = slice(t * RT, (t + 1) * RT)
        b2 = (keep[None, :] > keep[sl][:, None]) | (
            (keep[None, :] == keep[sl][:, None])
            & (rankf[None, :] < rankf[sl][:, None]))
        franks.append(jnp.sum(b2.astype(jnp.int32), axis=1))
    frank = jnp.concatenate(franks)
    iota_r = lax.broadcasted_iota(jnp.int32, (128, CAP), 0)
    Q = (frank[None, :] == iota_r).astype(jnp.float32)
    G = jnp.stack([mid[2], mid[3], mid[4], mid[5], keep, mid[6],
                   keep, keep], axis=1)
    out_ref[0] = jnp.dot(Q, G, precision=lax.Precision.HIGHEST)


def _nms(imgsz_f32, cand):
    mid = pl.pallas_call(
        _nms_a_body,
        grid=(B, CAP // RT),
        in_specs=[
            pl.BlockSpec(memory_space=pltpu.SMEM),
            pl.BlockSpec((1, NF, CAP), lambda b, t: (b, 0, 0)),
            pl.BlockSpec((1, NF, RT), lambda b, t: (b, 0, t)),
        ],
        out_specs=pl.BlockSpec((1, 8, RT), lambda b, t: (b, 0, t)),
        out_shape=jax.ShapeDtypeStruct((B, 8, CAP), jnp.float32),
    )(imgsz_f32, cand, cand)
    return pl.pallas_call(
        _nms_b_body,
        grid=(B,),
        in_specs=[pl.BlockSpec((1, 8, CAP), lambda b: (b, 0, 0))],
        out_specs=pl.BlockSpec((1, 128, 8), lambda b: (b, 0, 0)),
        out_shape=jax.ShapeDtypeStruct((B, 128, 8), jnp.float32),
    )(mid)


def _excl_cumsum(x, axis):
    c = jnp.cumsum(x, axis=axis)
    return c - x


def _pick_threshold(counts, need):
    nb = counts.shape[1]
    cum = jnp.cumsum(counts[:, ::-1], axis=1)[:, ::-1]
    ge = cum >= need[:, None]
    idx = jnp.max(jnp.where(ge, jnp.arange(nb, dtype=jnp.int32)[None, :], -1),
                  axis=1)
    cum_pad = jnp.concatenate([cum, jnp.zeros((B, 1), jnp.int32)], axis=1)
    ngt = jnp.take_along_axis(cum_pad, (idx + 1)[:, None], axis=1)[:, 0]
    return idx, ngt


def _build_src(cnts, m):
    ngt = cnts[:, 0].reshape(B, TPB)
    neq = jnp.minimum(cnts[:, 1].reshape(B, TPB), CAP)
    gt_off = _excl_cumsum(ngt, 1)
    total_gt = jnp.sum(ngt, axis=1)
    eq_take = jnp.clip(m[:, None] - _excl_cumsum(neq, 1), 0, neq)
    eq_off = _excl_cumsum(eq_take, 1)
    j = jnp.arange(CAP, dtype=jnp.int32)[None, :]
    j2 = j - total_gt[:, None]
    bb = jnp.arange(B, dtype=jnp.int32)[:, None]
    src = jnp.zeros((B, CAP), jnp.int32)
    for t in range(TPB):
        go = gt_off[:, t:t + 1]
        in_g = (j >= go) & (j < go + ngt[:, t:t + 1])
        src = jnp.where(in_g, (bb * TPB + t) * CAP + (j - go), src)
        eo = eq_off[:, t:t + 1]
        in_e = (j2 >= eo) & (j2 < eo + eq_take[:, t:t + 1]) & (j < TOPN)
        src = jnp.where(in_e, NTILES * CAP + (bb * TPB + t) * CAP + (j2 - eo),
                        src)
    return src


def kernel(location, cls_pred, box_pred, center_pred, image_sizes):
    scores = _scores(cls_pred, center_pred)
    scores_flat = scores.reshape(B * N)

    need = jnp.full((B,), TOPN, jnp.int32)
    hist1 = _hist_hi()(scores_flat)
    h1 = jnp.sum(hist1.reshape(B, TPB, 16384), axis=1)
    bstar, ngt_hi = _pick_threshold(h1, need)

    bstar16 = jnp.zeros((16,), jnp.int32).at[:B].set(bstar)
    hist2 = _hist_lo()(scores_flat, bstar16)
    h2 = jnp.sum(hist2.reshape(B, TPB, 65536), axis=1)
    vlo, ngt_lo = _pick_threshold(h2, need - ngt_hi)
    vstar = jnp.left_shift(bstar, 16) | vlo
    n_gt = ngt_hi + ngt_lo
    m = need - n_gt

    vstar16 = jnp.zeros((16,), jnp.int32).at[:B].set(vstar)
    gt_idx, eq_idx, cnts = _collect()(scores_flat, vstar16)
    src = _build_src(cnts, m)

    lists_cat = jnp.concatenate([gt_idx.reshape(-1), eq_idx.reshape(-1)])
    cand = _gather()(lists_cat, src.reshape(-1), scores_flat,
                     location.reshape(-1), box_pred.reshape(-1))

    imgsz = image_sizes.astype(jnp.float32).reshape(1, 2)
    raw = _nms(imgsz, cand)
    detections = raw[:, :100, :5]
    labels = raw[:, :100, 5].astype(jnp.int32)
    return detections, labels


# speedup vs baseline: 6.0766x; 1.1247x over previous
import functools
import jax
import jax.numpy as jnp
from jax import lax
from jax.experimental import pallas as pl
from jax.experimental.pallas import tpu as pltpu
from jax.experimental.pallas import tpu_sc as plsc

H = 100
W = 200
HW = H * W
C = 80
B = 8
N = HW * C
TOPN = 1000
NTILES = 32
TPB = NTILES // B
RNG = N // TPB
CHUNK = 16000
NCHUNK = RNG // CHUNK
UNROLL = 10
CAP = 1024
PSC = 2000

@functools.lru_cache(maxsize=None)
def _mesh():
    return plsc.VectorSubcoreMesh(core_axis_name="c", subcore_axis_name="s")


def _wid():
    return lax.axis_index("s") * 2 + lax.axis_index("c")


def _score_body(cls_ref, ctr_ref, out_ref):
    cls = jax.nn.sigmoid(cls_ref[0].reshape(C, HW))
    ctr = jax.nn.sigmoid(ctr_ref[0].reshape(1, HW))
    s = jnp.where(cls > 0.05, cls * ctr, 0.0)
    out_ref[0] = s.T


def _scores(cls_pred, center_pred):
    return pl.pallas_call(
        _score_body,
        grid=(B,),
        in_specs=[
            pl.BlockSpec((1, C, H, W), lambda b: (b, 0, 0, 0)),
            pl.BlockSpec((1, 1, H, W), lambda b: (b, 0, 0, 0)),
        ],
        out_specs=pl.BlockSpec((1, HW, C), lambda b: (b, 0, 0)),
        out_shape=jax.ShapeDtypeStruct((B, HW, C), jnp.float32),
    )(cls_pred, center_pred)


def _hist_hi_body(scores_hbm, hist_out, chunk_v, hist_v, histb_v, sem):
    wid = _wid()
    base = wid * RNG

    def zero(i, _):
        hist_v[pl.ds(i * 16, 16)] = jnp.zeros((16,), jnp.int32)
        histb_v[pl.ds(i * 16, 16)] = jnp.zeros((16,), jnp.int32)
        return 0
    lax.fori_loop(0, 16384 // 16, zero, 0)

    ones = jnp.ones((16,), jnp.int32)

    def chunk(k, _):
        pltpu.sync_copy(scores_hbm.at[pl.ds(base + k * CHUNK, CHUNK)], chunk_v)

        def inner(i, _):
            for u in range(UNROLL):
                v = chunk_v[pl.ds((i * UNROLL + u) * 16, 16)]
                bits = lax.bitcast_convert_type(v, jnp.int32)
                hi = lax.shift_right_logical(bits, 16)
                tgt = hist_v if u % 2 == 0 else histb_v
                plsc.addupdate_scatter(tgt, [hi], ones, mask=bits >= 0)
            return 0
        lax.fori_loop(0, CHUNK // 16 // UNROLL, inner, 0)
        return 0
    lax.fori_loop(0, NCHUNK, chunk, 0)

    def merge(i, _):
        sl = pl.ds(i * 16, 16)
        hist_v[sl] = hist_v[sl] + histb_v[sl]
        return 0
    lax.fori_loop(0, 16384 // 16, merge, 0)
    pltpu.sync_copy(hist_v, hist_out.at[wid])


@functools.lru_cache(maxsize=None)
def _hist_hi():
    return functools.partial(
        pl.kernel, mesh=_mesh(),
        compiler_params=pltpu.CompilerParams(
            needs_layout_passes=False, use_tc_tiling_on_sc=False),
        out_type=jax.ShapeDtypeStruct((NTILES, 16384), jnp.int32),
        scratch_types=[
            pltpu.VMEM((CHUNK,), jnp.float32),
            pltpu.VMEM((16384,), jnp.int32),
            pltpu.VMEM((16384,), jnp.int32),
            pltpu.SemaphoreType.DMA,
        ],
    )(_hist_hi_body)


def _sel_lane(vec16, lane):
    msk = lax.iota(jnp.int32, 16) == lane
    return jnp.max(jnp.where(msk, vec16, jnp.int32(-2147483648)))


def _hist_lo_body(scores_hbm, bstar_hbm, hist_out, chunk_v, hist_v, bst_v, sem):
    wid = _wid()
    base = wid * RNG
    b = wid // TPB
    pltpu.sync_copy(bstar_hbm, bst_v)
    bstar = _sel_lane(bst_v[...], b)

    def zero(i, _):
        hist_v[pl.ds(i * 16, 16)] = jnp.zeros((16,), jnp.int32)
        return 0
    lax.fori_loop(0, 65536 // 16, zero, 0)

    ones = jnp.ones((16,), jnp.int32)

    def chunk(k, _):
        pltpu.sync_copy(scores_hbm.at[pl.ds(base + k * CHUNK, CHUNK)], chunk_v)

        def inner(i, _):
            for u in range(UNROLL):
                v = chunk_v[pl.ds((i * UNROLL + u) * 16, 16)]
                bits = lax.bitcast_convert_type(v, jnp.int32)
                hi = lax.shift_right_logical(bits, 16)
                lo = jnp.bitwise_and(bits, jnp.int32(0xFFFF))
                plsc.addupdate_scatter(hist_v, [lo], ones, mask=hi == bstar)
            return 0
        lax.fori_loop(0, CHUNK // 16 // UNROLL, inner, 0)
        return 0
    lax.fori_loop(0, NCHUNK, chunk, 0)
    pltpu.sync_copy(hist_v, hist_out.at[wid])


@functools.lru_cache(maxsize=None)
def _hist_lo():
    return functools.partial(
        pl.kernel, mesh=_mesh(),
        compiler_params=pltpu.CompilerParams(
            needs_layout_passes=False, use_tc_tiling_on_sc=False),
        out_type=jax.ShapeDtypeStruct((NTILES, 65536), jnp.int32),
        scratch_types=[
            pltpu.VMEM((CHUNK,), jnp.float32),
            pltpu.VMEM((65536,), jnp.int32),
            pltpu.VMEM((16,), jnp.int32),
            pltpu.SemaphoreType.DMA,
        ],
    )(_hist_lo_body)


def _collect_body(scores_hbm, vstar_hbm, gt_out, eq_out, cnt_out,
                  chunk_v, gt_v, eq_v, vst_v, cnt_v, cntg_v, cnte_v, sem):
    wid = _wid()
    base = wid * RNG
    b = wid // TPB
    ibase = (wid % TPB) * RNG
    pltpu.sync_copy(vstar_hbm, vst_v)
    vstar = _sel_lane(vst_v[...], b)
    lanes = lax.iota(jnp.int32, 16)
    cntg_v[...] = jnp.zeros((16,), jnp.int32)
    cnte_v[...] = jnp.zeros((16,), jnp.int32)

    def zero(i, _):
        gt_v[pl.ds(i * 16, 16)] = jnp.zeros((16,), jnp.int32)
        eq_v[pl.ds(i * 16, 16)] = jnp.zeros((16,), jnp.int32)
        return 0
    lax.fori_loop(0, CAP // 16, zero, 0)

    def chunk(k, _):
        pltpu.sync_copy(scores_hbm.at[pl.ds(base + k * CHUNK, CHUNK)], chunk_v)

        def inner(i, _):
            acc = None
            for u in range(UNROLL):
                v = chunk_v[pl.ds((i * UNROLL + u) * 16, 16)]
                bits = lax.bitcast_convert_type(v, jnp.int32)
                m = bits >= vstar
                acc = m if acc is None else (acc | m)

            @pl.when(jnp.any(acc))
            def _():
                cg = cntg_v[...]
                ce = cnte_v[...]
                for u in range(UNROLL):
                    v = chunk_v[pl.ds((i * UNROLL + u) * 16, 16)]
                    bits = lax.bitcast_convert_type(v, jnp.int32)
                    gidx = (ibase + k * CHUNK + (i * UNROLL + u) * 16) + lanes
                    gm = bits > vstar
                    em = bits == vstar
                    gpos = cg + plsc.cumsum(gm.astype(jnp.int32)) - 1
                    plsc.store_scatter(gt_v, [gpos], gidx,
                                       mask=gm & (gpos < CAP))
                    epos = ce + plsc.cumsum(em.astype(jnp.int32)) - 1
                    plsc.store_scatter(eq_v, [epos], gidx,
                                       mask=em & (epos < CAP))
                    cg = cg + jnp.sum(gm.astype(jnp.int32))
                    ce = ce + jnp.sum(em.astype(jnp.int32))
                cntg_v[...] = cg
                cnte_v[...] = ce
            return 0
        lax.fori_loop(0, CHUNK // 16 // UNROLL, inner, 0)
        return 0

    lax.fori_loop(0, NCHUNK, chunk, 0)
    cgt = jnp.max(cntg_v[...])
    ceq = jnp.max(cnte_v[...])
    cnt_v[...] = jnp.where(lanes == 0, cgt, jnp.where(lanes == 1, ceq, 0))
    pltpu.sync_copy(gt_v, gt_out.at[wid])
    pltpu.sync_copy(eq_v, eq_out.at[wid])
    pltpu.sync_copy(cnt_v, cnt_out.at[wid])


@functools.lru_cache(maxsize=None)
def _collect():
    return functools.partial(
        pl.kernel, mesh=_mesh(),
        compiler_params=pltpu.CompilerParams(
            needs_layout_passes=False, use_tc_tiling_on_sc=False),
        out_type=(
            jax.ShapeDtypeStruct((NTILES, CAP), jnp.int32),
            jax.ShapeDtypeStruct((NTILES, CAP), jnp.int32),
            jax.ShapeDtypeStruct((NTILES, 16), jnp.int32),
        ),
        scratch_types=[
            pltpu.VMEM((CHUNK,), jnp.float32),
            pltpu.VMEM((CAP,), jnp.int32),
            pltpu.VMEM((CAP,), jnp.int32),
            pltpu.VMEM((16,), jnp.int32),
            pltpu.VMEM((16,), jnp.int32),
            pltpu.VMEM((16,), jnp.int32),
            pltpu.VMEM((16,), jnp.int32),
            pltpu.SemaphoreType.DMA,
        ],
    )(_collect_body)


SLOTS = CAP // TPB
NF = 16


def _gather_body(lists_hbm, src_hbm, scores_hbm, loc_hbm, box_hbm, out_hbm,
                 src_v, cand_v, addr_v, fld_v, sem):
    wid = _wid()
    b = wid // TPB
    q = wid % TPB
    pltpu.sync_copy(src_hbm.at[pl.ds(b * CAP + q * SLOTS, SLOTS)], src_v)
    pltpu.async_copy(lists_hbm.at[src_v], cand_v, sem).wait()

    def addrs(i, field, fn):
        def body(j, _):
            cand = cand_v[pl.ds(j * 16, 16)]
            addr_v[pl.ds(j * 16, 16)] = fn(cand)
            return 0
        lax.fori_loop(0, SLOTS // 16, body, 0)

    addrs(0, 0, lambda cand: b * N + cand)
    pltpu.async_copy(scores_hbm.at[addr_v], fld_v.at[0], sem).wait()
    addrs(0, 1, lambda cand: 2 * (cand // C))
    pltpu.async_copy(loc_hbm.at[addr_v], fld_v.at[1], sem).wait()
    addrs(0, 2, lambda cand: 2 * (cand // C) + 1)
    pltpu.async_copy(loc_hbm.at[addr_v], fld_v.at[2], sem).wait()
    for f in range(4):
        addrs(0, 3 + f, lambda cand, f=f: (b * 4 + f) * HW + cand // C)
        pltpu.async_copy(box_hbm.at[addr_v], fld_v.at[3 + f], sem).wait()

    def cls_body(j, _):
        cand = cand_v[pl.ds(j * 16, 16)]
        c = cand - (cand // C) * C
        fld_v[7, pl.ds(j * 16, 16)] = (c + 1).astype(jnp.float32)
        fld_v[8, pl.ds(j * 16, 16)] = cand.astype(jnp.float32)
        return 0
    lax.fori_loop(0, SLOTS // 16, cls_body, 0)

    for f in range(9):
        pltpu.sync_copy(fld_v.at[f], out_hbm.at[b, f, pl.ds(q * SLOTS, SLOTS)])


@functools.lru_cache(maxsize=None)
def _gather():
    return functools.partial(
        pl.kernel, mesh=_mesh(),
        compiler_params=pltpu.CompilerParams(
            needs_layout_passes=False, use_tc_tiling_on_sc=False),
        out_type=jax.ShapeDtypeStruct((B, NF, CAP), jnp.float32),
        scratch_types=[
            pltpu.VMEM((SLOTS,), jnp.int32),
            pltpu.VMEM((SLOTS,), jnp.int32),
            pltpu.VMEM((SLOTS,), jnp.int32),
            pltpu.VMEM((NF, SLOTS), jnp.float32),
            pltpu.SemaphoreType.DMA,
        ],
    )(_gather_body)


RT = 128


def _decode(cand, jbase, nlanes, wimg, himg):
    j_i32 = jbase + lax.broadcasted_iota(jnp.int32, (nlanes,), 0)
    jf = j_i32.astype(jnp.float32)
    padm = j_i32 >= TOPN
    score = jnp.where(padm, -1.0, cand[0])
    ridx = jnp.where(padm, 2.0e6 + jf, cand[8])
    s = jnp.where(padm, 0.0, cand[0])
    x1 = jnp.clip(cand[1] - cand[3], 0.0, wimg - 1.0)
    y1 = jnp.clip(cand[2] - cand[4], 0.0, himg - 1.0)
    x2 = jnp.clip(cand[1] + cand[5], 0.0, wimg - 1.0)
    y2 = jnp.clip(cand[2] + cand[6], 0.0, himg - 1.0)
    det = jnp.sqrt(jnp.maximum(s, 1e-12))
    valid = s > 0.0
    off = cand[7] * (jnp.maximum(wimg, himg) + 1.0)
    area = jnp.maximum(x2 - x1, 0.0) * jnp.maximum(y2 - y1, 0.0)
    return dict(score=score, ridx=ridx, det=det, valid=valid, x1=x1, y1=y1,
                x2=x2, y2=y2, ox1=x1 + off, oy1=y1 + off, ox2=x2 + off,
                oy2=y2 + off, area=area, cls=cand[7])


def _nms_a_body(sz_ref, cand_ref, ctile_ref, out_ref):
    t = pl.program_id(1)
    wimg = sz_ref[0, 1]
    himg = sz_ref[0, 0]
    F = _decode(cand_ref[0], 0, CAP, wimg, himg)
    T = _decode(ctile_ref[0], t * RT, RT, wimg, himg)

    before = (F["score"][None, :] > T["score"][:, None]) | (
        (F["score"][None, :] == T["score"][:, None])
        & (F["ridx"][None, :] < T["ridx"][:, None]))
    rank = jnp.sum(before.astype(jnp.int32), axis=1).astype(jnp.float32)

    ltx = jnp.maximum(T["ox1"][:, None], F["ox1"][None, :])
    lty = jnp.maximum(T["oy1"][:, None], F["oy1"][None, :])
    rbx = jnp.minimum(T["ox2"][:, None], F["ox2"][None, :])
    rby = jnp.minimum(T["oy2"][:, None], F["oy2"][None, :])
    ww = jnp.clip(rbx - ltx, 0.0, None)
    hh = jnp.clip(rby - lty, 0.0, None)
    inter = ww * hh
    union = T["area"][:, None] + F["area"][None, :] - inter
    iou = inter / jnp.maximum(union, 1e-6)
    hot = ((iou > 0.6) & (F["det"][None, :] > T["det"][:, None])
           & F["valid"][None, :])
    sup = jnp.any(hot, axis=1)

    keep = jnp.where(T["valid"] & (~sup), T["det"], 0.0)
    out_ref[0] = jnp.stack(
        [keep, rank, T["x1"], T["y1"], T["x2"], T["y2"],
         T["cls"], keep], axis=0)


def _nms_b_body(mid_ref, out_ref):
    mid = mid_ref[0]
    keep = mid[0]
    rankf = mid[1]
    franks = []
    for t in range(CAP // RT):
        sl = slice(t * RT, (t + 1) * RT)
        b2 = (keep[None, :] > keep[sl][:, None]) | (
            (keep[None, :] == keep[sl][:, None])
            & (rankf[None, :] < rankf[sl][:, None]))
        franks.append(jnp.sum(b2.astype(jnp.int32), axis=1))
    frank = jnp.concatenate(franks)
    iota_r = lax.broadcasted_iota(jnp.int32, (128, CAP), 0)
    Q = (frank[None, :] == iota_r).astype(jnp.float32)
    G = jnp.stack([mid[2], mid[3], mid[4], mid[5], keep, mid[6],
                   keep, keep], axis=1)
    out_ref[0] = jnp.dot(Q, G, precision=lax.Precision.HIGHEST)


def _nms(imgsz_f32, cand):
    mid = pl.pallas_call(
        _nms_a_body,
        grid=(B, CAP // RT),
        in_specs=[
            pl.BlockSpec(memory_space=pltpu.SMEM),
            pl.BlockSpec((1, NF, CAP), lambda b, t: (b, 0, 0)),
            pl.BlockSpec((1, NF, RT), lambda b, t: (b, 0, t)),
        ],
        out_specs=pl.BlockSpec((1, 8, RT), lambda b, t: (b, 0, t)),
        out_shape=jax.ShapeDtypeStruct((B, 8, CAP), jnp.float32),
    )(imgsz_f32, cand, cand)
    return pl.pallas_call(
        _nms_b_body,
        grid=(B,),
        in_specs=[pl.BlockSpec((1, 8, CAP), lambda b: (b, 0, 0))],
        out_specs=pl.BlockSpec((1, 128, 8), lambda b: (b, 0, 0)),
        out_shape=jax.ShapeDtypeStruct((B, 128, 8), jnp.float32),
    )(mid)


def _excl_cumsum(x, axis):
    c = jnp.cumsum(x, axis=axis)
    return c - x


def _pick_threshold(counts, need):
    nb = counts.shape[1]
    cum = jnp.cumsum(counts[:, ::-1], axis=1)[:, ::-1]
    ge = cum >= need[:, None]
    idx = jnp.max(jnp.where(ge, jnp.arange(nb, dtype=jnp.int32)[None, :], -1),
                  axis=1)
    cum_pad = jnp.concatenate([cum, jnp.zeros((B, 1), jnp.int32)], axis=1)
    ngt = jnp.take_along_axis(cum_pad, (idx + 1)[:, None], axis=1)[:, 0]
    return idx, ngt


def _build_src(cnts, m):
    ngt = cnts[:, 0].reshape(B, TPB)
    neq = jnp.minimum(cnts[:, 1].reshape(B, TPB), CAP)
    gt_off = _excl_cumsum(ngt, 1)
    total_gt = jnp.sum(ngt, axis=1)
    eq_take = jnp.clip(m[:, None] - _excl_cumsum(neq, 1), 0, neq)
    eq_off = _excl_cumsum(eq_take, 1)
    j = jnp.arange(CAP, dtype=jnp.int32)[None, :]
    j2 = j - total_gt[:, None]
    bb = jnp.arange(B, dtype=jnp.int32)[:, None]
    src = jnp.zeros((B, CAP), jnp.int32)
    for t in range(TPB):
        go = gt_off[:, t:t + 1]
        in_g = (j >= go) & (j < go + ngt[:, t:t + 1])
        src = jnp.where(in_g, (bb * TPB + t) * CAP + (j - go), src)
        eo = eq_off[:, t:t + 1]
        in_e = (j2 >= eo) & (j2 < eo + eq_take[:, t:t + 1]) & (j < TOPN)
        src = jnp.where(in_e, NTILES * CAP + (bb * TPB + t) * CAP + (j2 - eo),
                        src)
    return src


def kernel(location, cls_pred, box_pred, center_pred, image_sizes):
    scores = _scores(cls_pred, center_pred)
    scores_flat = scores.reshape(B * N)

    need = jnp.full((B,), TOPN, jnp.int32)
    hist1 = _hist_hi()(scores_flat)
    h1 = jnp.sum(hist1.reshape(B, TPB, 16384), axis=1)
    bstar, ngt_hi = _pick_threshold(h1, need)

    bstar16 = jnp.zeros((16,), jnp.int32).at[:B].set(bstar)
    hist2 = _hist_lo()(scores_flat, bstar16)
    h2 = jnp.sum(hist2.reshape(B, TPB, 65536), axis=1)
    vlo, ngt_lo = _pick_threshold(h2, need - ngt_hi)
    vstar = jnp.left_shift(bstar, 16) | vlo
    n_gt = ngt_hi + ngt_lo
    m = need - n_gt

    vstar16 = jnp.zeros((16,), jnp.int32).at[:B].set(vstar)
    gt_idx, eq_idx, cnts = _collect()(scores_flat, vstar16)
    src = _build_src(cnts, m)

    lists_cat = jnp.concatenate([gt_idx.reshape(-1), eq_idx.reshape(-1)])
    cand = _gather()(lists_cat, src.reshape(-1), scores_flat,
                     location.reshape(-1), box_pred.reshape(-1))

    imgsz = image_sizes.astype(jnp.float32).reshape(1, 2)
    raw = _nms(imgsz, cand)
    detections = raw[:, :100, :5]
    labels = raw[:, :100, 5].astype(jnp.int32)
    return detections, labels
